# baseline (speedup 1.0000x reference)
import functools

import jax
import jax.numpy as jnp
from jax.experimental import pallas as pl
from jax.experimental.pallas import tpu as pltpu


_NEG_INF = float("-inf")


def _dense_tail_body(a_ref, w3s_ref, b3_ref, gw1_ref, gb1_ref, gw2_ref,
                     gb2_ref, mw_ref, mb_ref, h_ref, g_ref,
                     m_s, s_s, g_s):
    i = pl.program_id(0)
    nblk = pl.num_programs(0)

    @pl.when(i == 0)
    def _init():
        m_s[0, 0] = _NEG_INF
        s_s[0, 0] = 0.0
        g_s[...] = jnp.zeros_like(g_s)

    a = a_ref[...]
    h = jnp.dot(a, w3s_ref[...], preferred_element_type=jnp.float32) * 0.25
    h = h + b3_ref[...]
    h_ref[...] = h

    z1 = jnp.maximum(
        jnp.dot(h, gw1_ref[...], preferred_element_type=jnp.float32)
        + gb1_ref[...], 0.0)
    z = jnp.dot(z1, gw2_ref[...], preferred_element_type=jnp.float32)
    z = z + gb2_ref[0, 0]

    blk_max = jnp.max(z)
    m_old = m_s[0, 0]
    m_new = jnp.maximum(m_old, blk_max)
    corr = jnp.exp(m_old - m_new)
    p = jnp.exp(z - m_new)
    s_s[0, 0] = s_s[0, 0] * corr + jnp.sum(p)
    g_s[...] = g_s[...] * corr + jnp.sum(p * h, axis=0, keepdims=True)
    m_s[0, 0] = m_new

    @pl.when(i == nblk - 1)
    def _fin():
        g = g_s[...] / s_s[0, 0]
        g = jnp.dot(g, mw_ref[...], preferred_element_type=jnp.float32)
        g_ref[...] = jnp.maximum(g + mb_ref[...], 0.0)


def _dense_tail(a, w3s, b3, gw1, gb1, gw2, gb2, mw, mb, blk=400):
    n, k = a.shape
    hid = w3s.shape[1]
    grid = n // blk
    return pl.pallas_call(
        _dense_tail_body,
        grid=(grid,),
        in_specs=[
            pl.BlockSpec((blk, k), lambda i: (i, 0)),
            pl.BlockSpec((k, hid), lambda i: (0, 0)),
            pl.BlockSpec((1, hid), lambda i: (0, 0)),
            pl.BlockSpec((hid, hid), lambda i: (0, 0)),
            pl.BlockSpec((1, hid), lambda i: (0, 0)),
            pl.BlockSpec((hid, 1), lambda i: (0, 0)),
            pl.BlockSpec((1, 1), lambda i: (0, 0)),
            pl.BlockSpec((hid, hid), lambda i: (0, 0)),
            pl.BlockSpec((1, hid), lambda i: (0, 0)),
        ],
        out_specs=[
            pl.BlockSpec((blk, hid), lambda i: (i, 0)),
            pl.BlockSpec((1, hid), lambda i: (0, 0)),
        ],
        out_shape=[
            jax.ShapeDtypeStruct((n, hid), jnp.float32),
            jax.ShapeDtypeStruct((1, hid), jnp.float32),
        ],
        scratch_shapes=[
            pltpu.SMEM((1, 1), jnp.float32),
            pltpu.SMEM((1, 1), jnp.float32),
            pltpu.VMEM((1, hid), jnp.float32),
        ],
    )(a, w3s, b3, gw1, gb1, gw2, gb2, mw, mb)


def kernel(x, edge_index, W1, b1, W2, b2, W3, att_src, att_dst, b3,
           gate_W1, gate_b1, gate_W2, gate_b2, mlp_W, mlp_b):
    n = x.shape[0]
    heads, hidden = att_src.shape
    in2 = W3.shape[0]

    loop = jnp.arange(n, dtype=edge_index.dtype)
    src = jnp.concatenate([edge_index[0], loop])
    dst = jnp.concatenate([edge_index[1], loop])

    deg = jax.ops.segment_sum(jnp.ones(src.shape[0], jnp.float32), dst,
                              num_segments=n)
    dinv = jax.lax.rsqrt(jnp.maximum(deg, 1e-12))

    xs = x * dinv[:, None]
    agg1 = jax.ops.segment_sum(xs[src], dst, num_segments=n)
    x1 = jnp.maximum((agg1 * dinv[:, None]) @ W1 + b1, 0.0)

    x1s = x1 * dinv[:, None]
    agg2 = jax.ops.segment_sum(x1s[src], dst, num_segments=n)
    x2 = jnp.maximum((agg2 * dinv[:, None]) @ W2 + b2, 0.0)

    w3r = W3.reshape(in2, heads, hidden)
    avs = jnp.einsum("khj,hj->kh", w3r, att_src)
    avd = jnp.einsum("khj,hj->kh", w3r, att_dst)
    a_s = x2 @ avs
    a_d = x2 @ avd
    c = jax.nn.leaky_relu(a_s + a_d, negative_slope=0.2)

    alpha = jax.nn.leaky_relu(a_s[src] + a_d[dst], negative_slope=0.2)
    ex = jnp.exp(alpha - c[dst])
    denom = jax.ops.segment_sum(ex, dst, num_segments=n)
    r = 1.0 / (denom + 1e-16)

    agg3 = jax.ops.segment_sum(ex[:, :, None] * x2[src][:, None, :], dst,
                               num_segments=n)
    a_flat = (agg3 * r[:, :, None]).reshape(n, heads * in2)

    w3s = w3r.transpose(1, 0, 2).reshape(heads * in2, hidden)
    h, g = _dense_tail(a_flat, w3s, b3.reshape(1, hidden),
                       gate_W1, gate_b1.reshape(1, hidden),
                       gate_W2, gate_b2.reshape(1, 1),
                       mlp_W, mlp_b.reshape(1, hidden))
    return (g, h)


# speedup vs baseline: 12.5293x; 12.5293x over previous
import functools

import jax
import jax.numpy as jnp
from jax import lax
from jax.experimental import pallas as pl
from jax.experimental.pallas import tpu as pltpu
from jax.experimental.pallas import tpu_sc as plsc

_NEG_INF = float("-inf")
_NPAD = 10240
_CH = 128


def _segsum_sc(vals, srci, dsti, zeros, f, etot):
    npad = _NPAD
    cpt = etot // (32 * _CH)
    ept = etot // 32
    rpt = npad // 16
    mesh = plsc.VectorSubcoreMesh(core_axis_name="c", subcore_axis_name="s")

    @functools.partial(
        pl.kernel,
        out_type=jax.ShapeDtypeStruct((2, npad, f), jnp.float32),
        mesh=mesh,
        scratch_types=[
            pltpu.VMEM((_CH,), jnp.int32),
            pltpu.VMEM((_CH,), jnp.int32),
            pltpu.VMEM((_CH, f), jnp.float32),
            pltpu.VMEM_SHARED((npad, f), jnp.float32),
            pltpu.SemaphoreType.DMA,
        ],
        compiler_params=pltpu.CompilerParams(use_tc_tiling_on_sc=False),
    )
    def k(vals_h, srci_h, dsti_h, zeros_h, out_h, sidx, didx, rows, acc, sem):
        cid = lax.axis_index("c")
        sid = lax.axis_index("s")
        wid = sid * 2 + cid
        pltpu.sync_copy(zeros_h.at[pl.ds(sid * rpt, rpt)],
                        acc.at[pl.ds(sid * rpt, rpt)])
        plsc.subcore_barrier()

        def body(j, carry):
            base = wid * ept + j * _CH
            pltpu.sync_copy(srci_h.at[pl.ds(base, _CH)], sidx)
            pltpu.sync_copy(dsti_h.at[pl.ds(base, _CH)], didx)
            pltpu.async_copy(vals_h.at[sidx], rows, sem).wait()
            pltpu.sync_copy(rows, acc.at[didx], add=True)
            return carry

        lax.fori_loop(0, cpt, body, 0)
        plsc.subcore_barrier()
        pltpu.sync_copy(acc.at[pl.ds(sid * rpt, rpt)],
                        out_h.at[cid, pl.ds(sid * rpt, rpt)])

    return k(vals, srci, dsti, zeros)


def _gat_edge_sc(asT, adT, cT, srci, dsti, zeros16, etot):
    npad = _NPAD
    cpt = etot // (32 * _CH)
    ept = etot // 32
    rpt = npad // 16
    mesh = plsc.VectorSubcoreMesh(core_axis_name="c", subcore_axis_name="s")

    @functools.partial(
        pl.kernel,
        out_type=[
            jax.ShapeDtypeStruct((etot, 16), jnp.float32),
            jax.ShapeDtypeStruct((2, npad, 16), jnp.float32),
        ],
        mesh=mesh,
        scratch_types=[
            pltpu.VMEM((_CH,), jnp.int32),
            pltpu.VMEM((_CH,), jnp.int32),
            pltpu.VMEM((_CH, 16), jnp.float32),
            pltpu.VMEM((_CH, 16), jnp.float32),
            pltpu.VMEM((_CH, 16), jnp.float32),
            pltpu.VMEM((_CH, 16), jnp.float32),
            pltpu.VMEM_SHARED((npad, 16), jnp.float32),
            pltpu.SemaphoreType.DMA,
        ],
        compiler_params=pltpu.CompilerParams(use_tc_tiling_on_sc=False),
    )
    def k(as_h, ad_h, c_h, srci_h, dsti_h, zeros_h, ex_h, dn_h,
          sidx, didx, ar, br, cr, exb, acc, sem):
        cid = lax.axis_index("c")
        sid = lax.axis_index("s")
        wid = sid * 2 + cid
        pltpu.sync_copy(zeros_h.at[pl.ds(sid * rpt, rpt)],
                        acc.at[pl.ds(sid * rpt, rpt)])
        plsc.subcore_barrier()
        mask = lax.iota(jnp.int32, 16) < 4

        def body(j, carry):
            base = wid * ept + j * _CH
            pltpu.sync_copy(srci_h.at[pl.ds(base, _CH)], sidx)
            pltpu.sync_copy(dsti_h.at[pl.ds(base, _CH)], didx)
            pltpu.async_copy(as_h.at[sidx], ar, sem).wait()
            pltpu.async_copy(ad_h.at[didx], br, sem).wait()
            pltpu.async_copy(c_h.at[didx], cr, sem).wait()

            def inner(e, c2):
                pre = ar[e, :] + br[e, :]
                alpha = jnp.maximum(pre, 0.2 * pre)
                exv = jnp.exp(alpha - cr[e, :])
                exb[e, :] = jnp.where(mask, exv, 0.0)
                return c2

            lax.fori_loop(0, _CH, inner, 0)
            pltpu.sync_copy(exb, ex_h.at[pl.ds(base, _CH)])
            pltpu.sync_copy(exb, acc.at[didx], add=True)
            return carry

        lax.fori_loop(0, cpt, body, 0)
        plsc.subcore_barrier()
        pltpu.sync_copy(acc.at[pl.ds(sid * rpt, rpt)],
                        dn_h.at[cid, pl.ds(sid * rpt, rpt)])

    return k(asT, adT, cT, srci, dsti, zeros16)


def _gat_agg_sc(x2T, ex, srci, dsti, zeros128, etot):
    npad = _NPAD
    ept = etot // 16
    cpt = ept // _CH
    rpt = npad // 16
    mesh = plsc.VectorSubcoreMesh(core_axis_name="c", subcore_axis_name="s")

    @functools.partial(
        pl.kernel,
        out_type=jax.ShapeDtypeStruct((4, npad, 128), jnp.float32),
        mesh=mesh,
        scratch_types=[
            pltpu.VMEM((_CH,), jnp.int32),
            pltpu.VMEM((_CH,), jnp.int32),
            pltpu.VMEM((_CH, 128), jnp.float32),
            pltpu.VMEM((_CH, 16), jnp.float32),
            pltpu.VMEM((_CH, 128), jnp.float32),
            pltpu.VMEM_SHARED((npad, 128), jnp.float32),
            pltpu.SemaphoreType.DMA,
        ],
        compiler_params=pltpu.CompilerParams(use_tc_tiling_on_sc=False),
    )
    def k(x2_h, ex_h, srci_h, dsti_h, zeros_h, out_h,
          sidx, didx, xg, exg, scb, acc, sem):
        cid = lax.axis_index("c")
        sid = lax.axis_index("s")
        for hp in range(2):
            hh = cid * 2 + hp
            pltpu.sync_copy(zeros_h.at[pl.ds(sid * rpt, rpt)],
                            acc.at[pl.ds(sid * rpt, rpt)])
            plsc.subcore_barrier()

            def body(j, carry):
                base = sid * ept + j * _CH
                pltpu.sync_copy(srci_h.at[pl.ds(base, _CH)], sidx)
                pltpu.sync_copy(dsti_h.at[pl.ds(base, _CH)], didx)
                pltpu.async_copy(x2_h.at[sidx], xg, sem).wait()
                pltpu.sync_copy(ex_h.at[pl.ds(base, _CH)], exg)

                def inner(e, c2):
                    exv = exg[e, :]
                    w = jnp.where(cid == 0, exv[hp], exv[2 + hp])
                    for fb in range(8):
                        scb[e, pl.ds(fb * 16, 16)] = (
                            xg[e, pl.ds(fb * 16, 16)] * w)
                    return c2

                lax.fori_loop(0, _CH, inner, 0)
                pltpu.sync_copy(scb, acc.at[didx], add=True)
                return carry

            lax.fori_loop(0, cpt, body, 0)
            plsc.subcore_barrier()
            pltpu.sync_copy(acc.at[pl.ds(sid * rpt, rpt)],
                            out_h.at[hh, pl.ds(sid * rpt, rpt)])
            plsc.subcore_barrier()

    return k(x2T, ex, srci, dsti, zeros128)


def _gcn_dense_body(p_ref, dinv_ref, w_ref, b_ref, out_ref):
    s = (p_ref[0] + p_ref[1]) * dinv_ref[...]
    h = jnp.dot(s, w_ref[...], preferred_element_type=jnp.float32)
    h = jnp.maximum(h + b_ref[...], 0.0)
    out_ref[...] = h * dinv_ref[...]


def _gcn_dense(p, dinv, w, b, blk=1280):
    npad, fin = p.shape[1], p.shape[2]
    fout = w.shape[1]
    return pl.pallas_call(
        _gcn_dense_body,
        grid=(npad // blk,),
        in_specs=[
            pl.BlockSpec((2, blk, fin), lambda i: (0, i, 0)),
            pl.BlockSpec((blk, 1), lambda i: (i, 0)),
            pl.BlockSpec((fin, fout), lambda i: (0, 0)),
            pl.BlockSpec((1, fout), lambda i: (0, 0)),
        ],
        out_specs=pl.BlockSpec((blk, fout), lambda i: (i, 0)),
        out_shape=jax.ShapeDtypeStruct((npad, fout), jnp.float32),
    )(p, dinv, w, b)


def _gat_prep_body(p_ref, dinv_ref, w_ref, b_ref, avs_ref, avd_ref,
                   x2_ref, as_ref, ad_ref, c_ref):
    s = (p_ref[0] + p_ref[1]) * dinv_ref[...]
    x2 = jnp.dot(s, w_ref[...], preferred_element_type=jnp.float32)
    x2 = jnp.maximum(x2 + b_ref[...], 0.0)
    x2_ref[...] = x2
    a_s = jnp.dot(x2, avs_ref[...], preferred_element_type=jnp.float32)
    a_d = jnp.dot(x2, avd_ref[...], preferred_element_type=jnp.float32)
    pre = a_s + a_d
    c = jnp.maximum(pre, 0.2 * pre)
    z = jnp.zeros((a_s.shape[0], 12), jnp.float32)
    as_ref[...] = jnp.concatenate([a_s, z], axis=1)
    ad_ref[...] = jnp.concatenate([a_d, z], axis=1)
    c_ref[...] = jnp.concatenate([c, z], axis=1)


def _gat_prep(p, dinv, w, b, avs, avd, blk=1280):
    npad, fin = p.shape[1], p.shape[2]
    fout = w.shape[1]
    return pl.pallas_call(
        _gat_prep_body,
        grid=(npad // blk,),
        in_specs=[
            pl.BlockSpec((2, blk, fin), lambda i: (0, i, 0)),
            pl.BlockSpec((blk, 1), lambda i: (i, 0)),
            pl.BlockSpec((fin, fout), lambda i: (0, 0)),
            pl.BlockSpec((1, fout), lambda i: (0, 0)),
            pl.BlockSpec((fout, 4), lambda i: (0, 0)),
            pl.BlockSpec((fout, 4), lambda i: (0, 0)),
        ],
        out_specs=[
            pl.BlockSpec((blk, fout), lambda i: (i, 0)),
            pl.BlockSpec((blk, 16), lambda i: (i, 0)),
            pl.BlockSpec((blk, 16), lambda i: (i, 0)),
            pl.BlockSpec((blk, 16), lambda i: (i, 0)),
        ],
        out_shape=[
            jax.ShapeDtypeStruct((npad, fout), jnp.float32),
            jax.ShapeDtypeStruct((npad, 16), jnp.float32),
            jax.ShapeDtypeStruct((npad, 16), jnp.float32),
            jax.ShapeDtypeStruct((npad, 16), jnp.float32),
        ],
    )(p, dinv, w, b, avs, avd)


def _tail_body(a3_ref, dp_ref, w3s_ref, b3_ref, gw1_ref, gb1_ref, gw2_ref,
               gb2_ref, mw_ref, mb_ref, h_ref, g_ref, m_s, s_s, g_s):
    i = pl.program_id(0)
    nblk = pl.num_programs(0)

    @pl.when(i == 0)
    def _init():
        m_s[0, 0] = _NEG_INF
        s_s[0, 0] = 0.0
        g_s[...] = jnp.zeros_like(g_s)

    d = dp_ref[0] + dp_ref[1]
    r = 1.0 / (d + 1e-16)
    hid = h_ref.shape[1]
    acc = jnp.zeros((a3_ref.shape[1], hid), jnp.float32)
    for hh in range(4):
        sc = a3_ref[hh] * r[:, hh:hh + 1]
        acc = acc + jnp.dot(sc, w3s_ref[hh * 128:(hh + 1) * 128, :],
                            preferred_element_type=jnp.float32)
    h = acc * 0.25 + b3_ref[...]
    h_ref[...] = h

    z1 = jnp.maximum(
        jnp.dot(h, gw1_ref[...], preferred_element_type=jnp.float32)
        + gb1_ref[...], 0.0)
    z = jnp.dot(z1, gw2_ref[...], preferred_element_type=jnp.float32)
    z = z + gb2_ref[0, 0]

    blk_max = jnp.max(z)
    m_old = m_s[0, 0]
    m_new = jnp.maximum(m_old, blk_max)
    corr = jnp.exp(m_old - m_new)
    p = jnp.exp(z - m_new)
    s_s[0, 0] = s_s[0, 0] * corr + jnp.sum(p)
    g_s[...] = g_s[...] * corr + jnp.sum(p * h, axis=0, keepdims=True)
    m_s[0, 0] = m_new

    @pl.when(i == nblk - 1)
    def _fin():
        g = g_s[...] / s_s[0, 0]
        g = jnp.dot(g, mw_ref[...], preferred_element_type=jnp.float32)
        g_ref[...] = jnp.maximum(g + mb_ref[...], 0.0)


def _tail(a3, dp, w3s, b3, gw1, gb1, gw2, gb2, mw, mb, n, blk=400):
    hid = w3s.shape[1]
    return pl.pallas_call(
        _tail_body,
        grid=(n // blk,),
        in_specs=[
            pl.BlockSpec((4, blk, 128), lambda i: (0, i, 0)),
            pl.BlockSpec((2, blk, 16), lambda i: (0, i, 0)),
            pl.BlockSpec((512, hid), lambda i: (0, 0)),
            pl.BlockSpec((1, hid), lambda i: (0, 0)),
            pl.BlockSpec((hid, hid), lambda i: (0, 0)),
            pl.BlockSpec((1, hid), lambda i: (0, 0)),
            pl.BlockSpec((hid, 1), lambda i: (0, 0)),
            pl.BlockSpec((1, 1), lambda i: (0, 0)),
            pl.BlockSpec((hid, hid), lambda i: (0, 0)),
            pl.BlockSpec((1, hid), lambda i: (0, 0)),
        ],
        out_specs=[
            pl.BlockSpec((blk, hid), lambda i: (i, 0)),
            pl.BlockSpec((1, hid), lambda i: (0, 0)),
        ],
        out_shape=[
            jax.ShapeDtypeStruct((n, hid), jnp.float32),
            jax.ShapeDtypeStruct((1, hid), jnp.float32),
        ],
        scratch_shapes=[
            pltpu.SMEM((1, 1), jnp.float32),
            pltpu.SMEM((1, 1), jnp.float32),
            pltpu.VMEM((1, hid), jnp.float32),
        ],
    )(a3, dp, w3s, b3, gw1, gb1, gw2, gb2, mw, mb)


def kernel(x, edge_index, W1, b1, W2, b2, W3, att_src, att_dst, b3,
           gate_W1, gate_b1, gate_W2, gate_b2, mlp_W, mlp_b):
    n = x.shape[0]
    heads, hidden = att_src.shape
    in2 = W3.shape[0]
    npad = _NPAD

    loop = jnp.arange(n, dtype=jnp.int32)
    src = jnp.concatenate([edge_index[0], loop])
    dst = jnp.concatenate([edge_index[1], loop])
    e_real = src.shape[0]
    etot = ((e_real + 4095) // 4096) * 4096
    padn = etot - e_real
    srcp = jnp.concatenate([src, jnp.zeros((padn,), jnp.int32)])
    dstp = jnp.concatenate([dst, jnp.full((padn,), n, jnp.int32)])

    zeros16 = jnp.zeros((npad, 16), jnp.float32)
    zeros64 = jnp.zeros((npad, 64), jnp.float32)
    ones16 = jnp.ones((npad, 16), jnp.float32)

    degp = _segsum_sc(ones16, dstp, dstp, zeros16, 16, etot)
    deg = degp[0, :, 0] + degp[1, :, 0]
    dinv = lax.rsqrt(jnp.maximum(deg, 1e-12))
    dinv2 = dinv[:, None]

    xs16 = jnp.zeros((npad, 16), jnp.float32)
    xs16 = xs16.at[:n, :3].set(x * dinv2[:n])
    agg1 = _segsum_sc(xs16, srcp, dstp, zeros16, 16, etot)
    w1p = jnp.zeros((16, 64), jnp.float32).at[:3].set(W1)
    x1s = _gcn_dense(agg1, dinv2, w1p, b1.reshape(1, 64))

    agg2 = _segsum_sc(x1s, srcp, dstp, zeros64, 64, etot)
    w3r = W3.reshape(in2, heads, hidden)
    avs = jnp.einsum("khj,hj->kh", w3r, att_src)
    avd = jnp.einsum("khj,hj->kh", w3r, att_dst)
    x2, asT, adT, cT = _gat_prep(agg2, dinv2, W2, b2.reshape(1, in2),
                                 avs, avd)

    exT, dpart = _gat_edge_sc(asT, adT, cT, srcp, dstp, zeros16, etot)
    zeros128 = jnp.zeros((npad, 128), jnp.float32)
    a3 = _gat_agg_sc(x2, exT, srcp, dstp, zeros128, etot)

    w3s = w3r.transpose(1, 0, 2).reshape(heads * in2, hidden)
    h, g = _tail(a3, dpart, w3s, b3.reshape(1, hidden),
                 gate_W1, gate_b1.reshape(1, hidden),
                 gate_W2, gate_b2.reshape(1, 1),
                 mlp_W, mlp_b.reshape(1, hidden), n)
    return (g, h)


# speedup vs baseline: 18.2378x; 1.4556x over previous
import functools

import jax
import jax.numpy as jnp
from jax import lax
from jax.experimental import pallas as pl
from jax.experimental.pallas import tpu as pltpu
from jax.experimental.pallas import tpu_sc as plsc

_NEG_INF = float("-inf")
_NPAD = 10240
_CH = 128


def _segsum_sc(vals, srci, dsti, zeros, f, etot):
    npad = _NPAD
    cpt = etot // (32 * _CH)
    ept = etot // 32
    rpt = npad // 16
    mesh = plsc.VectorSubcoreMesh(core_axis_name="c", subcore_axis_name="s")

    @functools.partial(
        pl.kernel,
        out_type=jax.ShapeDtypeStruct((2, npad, f), jnp.float32),
        mesh=mesh,
        scratch_types=[
            pltpu.VMEM((_CH,), jnp.int32),
            pltpu.VMEM((_CH,), jnp.int32),
            pltpu.VMEM((_CH, f), jnp.float32),
            pltpu.VMEM_SHARED((npad, f), jnp.float32),
            pltpu.SemaphoreType.DMA,
        ],
        compiler_params=pltpu.CompilerParams(use_tc_tiling_on_sc=False),
    )
    def k(vals_h, srci_h, dsti_h, zeros_h, out_h, sidx, didx, rows, acc, sem):
        cid = lax.axis_index("c")
        sid = lax.axis_index("s")
        wid = sid * 2 + cid
        pltpu.sync_copy(zeros_h.at[pl.ds(sid * rpt, rpt)],
                        acc.at[pl.ds(sid * rpt, rpt)])
        plsc.subcore_barrier()

        def body(j, carry):
            base = wid * ept + j * _CH
            pltpu.sync_copy(srci_h.at[pl.ds(base, _CH)], sidx)
            pltpu.sync_copy(dsti_h.at[pl.ds(base, _CH)], didx)
            pltpu.async_copy(vals_h.at[sidx], rows, sem).wait()
            pltpu.sync_copy(rows, acc.at[didx], add=True)
            return carry

        lax.fori_loop(0, cpt, body, 0)
        plsc.subcore_barrier()
        pltpu.sync_copy(acc.at[pl.ds(sid * rpt, rpt)],
                        out_h.at[cid, pl.ds(sid * rpt, rpt)])

    return k(vals, srci, dsti, zeros)


def _deg_sc(dsti, ones_rows, zeros, etot):
    npad = _NPAD
    cpt = etot // (32 * _CH)
    ept = etot // 32
    rpt = npad // 16
    mesh = plsc.VectorSubcoreMesh(core_axis_name="c", subcore_axis_name="s")

    @functools.partial(
        pl.kernel,
        out_type=jax.ShapeDtypeStruct((2, npad, 16), jnp.float32),
        mesh=mesh,
        scratch_types=[
            pltpu.VMEM((_CH,), jnp.int32),
            pltpu.VMEM((_CH, 16), jnp.float32),
            pltpu.VMEM_SHARED((npad, 16), jnp.float32),
        ],
        compiler_params=pltpu.CompilerParams(use_tc_tiling_on_sc=False),
    )
    def k(dsti_h, ones_h, zeros_h, out_h, didx, rows, acc):
        cid = lax.axis_index("c")
        sid = lax.axis_index("s")
        wid = sid * 2 + cid
        pltpu.sync_copy(ones_h, rows)
        pltpu.sync_copy(zeros_h.at[pl.ds(sid * rpt, rpt)],
                        acc.at[pl.ds(sid * rpt, rpt)])
        plsc.subcore_barrier()

        def body(j, carry):
            base = wid * ept + j * _CH
            pltpu.sync_copy(dsti_h.at[pl.ds(base, _CH)], didx)
            pltpu.sync_copy(rows, acc.at[didx], add=True)
            return carry

        lax.fori_loop(0, cpt, body, 0)
        plsc.subcore_barrier()
        pltpu.sync_copy(acc.at[pl.ds(sid * rpt, rpt)],
                        out_h.at[cid, pl.ds(sid * rpt, rpt)])

    return k(dsti, ones_rows, zeros)


def _gat_edge_sc(asT, adT, cT, srci, dsti, zeros16, etot):
    npad = _NPAD
    cpt = etot // (32 * _CH)
    ept = etot // 32
    rpt = npad // 16
    mesh = plsc.VectorSubcoreMesh(core_axis_name="c", subcore_axis_name="s")

    @functools.partial(
        pl.kernel,
        out_type=[
            jax.ShapeDtypeStruct((etot, 16), jnp.float32),
            jax.ShapeDtypeStruct((2, npad, 16), jnp.float32),
        ],
        mesh=mesh,
        scratch_types=[
            pltpu.VMEM((_CH,), jnp.int32),
            pltpu.VMEM((_CH,), jnp.int32),
            pltpu.VMEM((_CH, 16), jnp.float32),
            pltpu.VMEM((_CH, 16), jnp.float32),
            pltpu.VMEM((_CH, 16), jnp.float32),
            pltpu.VMEM((_CH, 16), jnp.float32),
            pltpu.VMEM_SHARED((npad, 16), jnp.float32),
            pltpu.SemaphoreType.DMA,
            pltpu.SemaphoreType.DMA,
            pltpu.SemaphoreType.DMA,
        ],
        compiler_params=pltpu.CompilerParams(use_tc_tiling_on_sc=False),
    )
    def k(as_h, ad_h, c_h, srci_h, dsti_h, zeros_h, ex_h, dn_h,
          sidx, didx, ar, br, cr, exb, acc, sem, sem2, sem3):
        cid = lax.axis_index("c")
        sid = lax.axis_index("s")
        wid = sid * 2 + cid
        pltpu.sync_copy(zeros_h.at[pl.ds(sid * rpt, rpt)],
                        acc.at[pl.ds(sid * rpt, rpt)])
        plsc.subcore_barrier()
        mask = lax.iota(jnp.int32, 16) < 4

        def body(j, carry):
            base = wid * ept + j * _CH
            pltpu.sync_copy(srci_h.at[pl.ds(base, _CH)], sidx)
            pltpu.sync_copy(dsti_h.at[pl.ds(base, _CH)], didx)
            pltpu.async_copy(as_h.at[sidx], ar, sem)
            pltpu.async_copy(ad_h.at[didx], br, sem2)
            pltpu.async_copy(c_h.at[didx], cr, sem3)
            pltpu.make_async_copy(as_h.at[sidx], ar, sem).wait()
            pltpu.make_async_copy(ad_h.at[didx], br, sem2).wait()
            pltpu.make_async_copy(c_h.at[didx], cr, sem3).wait()

            def inner(e, c2):
                pre = ar[e, :] + br[e, :]
                alpha = jnp.maximum(pre, 0.2 * pre)
                exv = jnp.exp(alpha - cr[e, :])
                exb[e, :] = jnp.where(mask, exv, 0.0)
                return c2

            lax.fori_loop(0, _CH, inner, 0)
            pltpu.sync_copy(exb, ex_h.at[pl.ds(base, _CH)])
            pltpu.sync_copy(exb, acc.at[didx], add=True)
            return carry

        lax.fori_loop(0, cpt, body, 0)
        plsc.subcore_barrier()
        pltpu.sync_copy(acc.at[pl.ds(sid * rpt, rpt)],
                        dn_h.at[cid, pl.ds(sid * rpt, rpt)])

    return k(asT, adT, cT, srci, dsti, zeros16)


def _gat_agg_sc(xq0, xq1, xq2, xq3, ex, srci, dsti, zeros128, etot):
    npad = _NPAD
    ept = etot // 16
    cpt = ept // _CH
    rpt = npad // 16
    mesh = plsc.VectorSubcoreMesh(core_axis_name="c", subcore_axis_name="s")

    @functools.partial(
        pl.kernel,
        out_type=jax.ShapeDtypeStruct((2, 2, npad, 128), jnp.float32),
        mesh=mesh,
        scratch_types=[
            pltpu.VMEM((_CH,), jnp.int32),
            pltpu.VMEM((_CH,), jnp.int32),
            pltpu.VMEM((_CH,), jnp.int32),
            pltpu.VMEM((_CH,), jnp.int32),
            pltpu.VMEM((_CH, 32), jnp.float32),
            pltpu.VMEM((_CH, 32), jnp.float32),
            pltpu.VMEM((_CH, 16), jnp.float32),
            pltpu.VMEM((_CH, 128), jnp.float32),
            pltpu.VMEM_SHARED((npad, 128), jnp.float32),
            pltpu.SemaphoreType.DMA,
            pltpu.SemaphoreType.DMA,
        ],
        compiler_params=pltpu.CompilerParams(use_tc_tiling_on_sc=False),
    )
    def k(x0_h, x1_h, x2_h, x3_h, ex_h, srci_h, dsti_h, zeros_h, out_h,
          sidx0, sidx1, didx0, didx1, xg0, xg1, exg, scb, acc, sem0, sem1):
        cid = lax.axis_index("c")
        sid = lax.axis_index("s")
        sidxs = (sidx0, sidx1)
        didxs = (didx0, didx1)
        xgs = (xg0, xg1)
        sems = (sem0, sem1)

        for p in range(2):
            xq_h = (x0_h, x2_h) if p == 0 else (x1_h, x3_h)
            pltpu.sync_copy(zeros_h.at[pl.ds(sid * rpt, rpt)],
                            acc.at[pl.ds(sid * rpt, rpt)])
            plsc.subcore_barrier()

            def load_idx(j, b):
                base = sid * ept + j * _CH
                pltpu.sync_copy(srci_h.at[pl.ds(base, _CH)], sidxs[b])
                pltpu.sync_copy(dsti_h.at[pl.ds(base, _CH)], didxs[b])

            def issue(b, q0_h, q1_h):
                @pl.when(cid == 0)
                def _a():
                    pltpu.async_copy(q0_h.at[sidxs[b]], xgs[b], sems[b])

                @pl.when(cid == 1)
                def _b():
                    pltpu.async_copy(q1_h.at[sidxs[b]], xgs[b], sems[b])

            def wait(b, q0_h, q1_h):
                @pl.when(cid == 0)
                def _a():
                    pltpu.make_async_copy(q0_h.at[sidxs[b]], xgs[b],
                                          sems[b]).wait()

                @pl.when(cid == 1)
                def _b():
                    pltpu.make_async_copy(q1_h.at[sidxs[b]], xgs[b],
                                          sems[b]).wait()

            q0_h, q1_h = xq_h
            load_idx(0, 0)
            issue(0, q0_h, q1_h)
            load_idx(1, 1)
            issue(1, q0_h, q1_h)

            def body(j2, carry):
                for b in range(2):
                    j = j2 * 2 + b
                    base = sid * ept + j * _CH
                    wait(b, q0_h, q1_h)
                    pltpu.sync_copy(ex_h.at[pl.ds(base, _CH)], exg)
                    xg = xgs[b]

                    def inner(e, c2):
                        exv = exg[e, :]
                        v0 = xg[e, pl.ds(0, 16)]
                        v1 = xg[e, pl.ds(16, 16)]
                        for hh in range(4):
                            w = exv[hh]
                            scb[e, pl.ds(hh * 32, 16)] = v0 * w
                            scb[e, pl.ds(hh * 32 + 16, 16)] = v1 * w
                        return c2

                    lax.fori_loop(0, _CH, inner, 0)
                    pltpu.sync_copy(scb, acc.at[didxs[b]], add=True)

                    @pl.when(j + 2 < cpt)
                    def _next():
                        load_idx(j + 2, b)
                        issue(b, q0_h, q1_h)
                return carry

            lax.fori_loop(0, cpt // 2, body, 0)
            plsc.subcore_barrier()
            pltpu.sync_copy(acc.at[pl.ds(sid * rpt, rpt)],
                            out_h.at[cid, p, pl.ds(sid * rpt, rpt)])
            plsc.subcore_barrier()

    return k(xq0, xq1, xq2, xq3, ex, srci, dsti, zeros128)


def _gcn_dense_body(p_ref, dinv_ref, w_ref, b_ref, out_ref):
    s = (p_ref[0] + p_ref[1]) * dinv_ref[...]
    h = jnp.dot(s, w_ref[...], preferred_element_type=jnp.float32)
    h = jnp.maximum(h + b_ref[...], 0.0)
    out_ref[...] = h * dinv_ref[...]


def _gcn_dense(p, dinv, w, b, blk=1280):
    npad, fin = p.shape[1], p.shape[2]
    fout = w.shape[1]
    return pl.pallas_call(
        _gcn_dense_body,
        grid=(npad // blk,),
        in_specs=[
            pl.BlockSpec((2, blk, fin), lambda i: (0, i, 0)),
            pl.BlockSpec((blk, 1), lambda i: (i, 0)),
            pl.BlockSpec((fin, fout), lambda i: (0, 0)),
            pl.BlockSpec((1, fout), lambda i: (0, 0)),
        ],
        out_specs=pl.BlockSpec((blk, fout), lambda i: (i, 0)),
        out_shape=jax.ShapeDtypeStruct((npad, fout), jnp.float32),
    )(p, dinv, w, b)


def _gat_prep_body(p_ref, dinv_ref, w_ref, b_ref, avs_ref, avd_ref,
                   x2_ref, as_ref, ad_ref, c_ref):
    s = (p_ref[0] + p_ref[1]) * dinv_ref[...]
    x2 = jnp.dot(s, w_ref[...], preferred_element_type=jnp.float32)
    x2 = jnp.maximum(x2 + b_ref[...], 0.0)
    x2_ref[...] = x2
    a_s = jnp.dot(x2, avs_ref[...], preferred_element_type=jnp.float32)
    a_d = jnp.dot(x2, avd_ref[...], preferred_element_type=jnp.float32)
    pre = a_s + a_d
    c = jnp.maximum(pre, 0.2 * pre)
    z = jnp.zeros((a_s.shape[0], 12), jnp.float32)
    as_ref[...] = jnp.concatenate([a_s, z], axis=1)
    ad_ref[...] = jnp.concatenate([a_d, z], axis=1)
    c_ref[...] = jnp.concatenate([c, z], axis=1)


def _gat_prep(p, dinv, w, b, avs, avd, blk=1280):
    npad, fin = p.shape[1], p.shape[2]
    fout = w.shape[1]
    return pl.pallas_call(
        _gat_prep_body,
        grid=(npad // blk,),
        in_specs=[
            pl.BlockSpec((2, blk, fin), lambda i: (0, i, 0)),
            pl.BlockSpec((blk, 1), lambda i: (i, 0)),
            pl.BlockSpec((fin, fout), lambda i: (0, 0)),
            pl.BlockSpec((1, fout), lambda i: (0, 0)),
            pl.BlockSpec((fout, 4), lambda i: (0, 0)),
            pl.BlockSpec((fout, 4), lambda i: (0, 0)),
        ],
        out_specs=[
            pl.BlockSpec((blk, fout), lambda i: (i, 0)),
            pl.BlockSpec((blk, 16), lambda i: (i, 0)),
            pl.BlockSpec((blk, 16), lambda i: (i, 0)),
            pl.BlockSpec((blk, 16), lambda i: (i, 0)),
        ],
        out_shape=[
            jax.ShapeDtypeStruct((npad, fout), jnp.float32),
            jax.ShapeDtypeStruct((npad, 16), jnp.float32),
            jax.ShapeDtypeStruct((npad, 16), jnp.float32),
            jax.ShapeDtypeStruct((npad, 16), jnp.float32),
        ],
    )(p, dinv, w, b, avs, avd)


def _tail_body(a3_ref, dp_ref, w3s_ref, b3_ref, gw1_ref, gb1_ref, gw2_ref,
               gb2_ref, mw_ref, mb_ref, h_ref, g_ref, m_s, s_s, g_s):
    i = pl.program_id(0)
    nblk = pl.num_programs(0)

    @pl.when(i == 0)
    def _init():
        m_s[0, 0] = _NEG_INF
        s_s[0, 0] = 0.0
        g_s[...] = jnp.zeros_like(g_s)

    d = dp_ref[0] + dp_ref[1]
    r = 1.0 / (d + 1e-16)
    hid = h_ref.shape[1]
    acc = jnp.zeros((a3_ref.shape[1], hid), jnp.float32)
    for hh in range(4):
        sc = a3_ref[hh] * r[:, hh:hh + 1]
        acc = acc + jnp.dot(sc, w3s_ref[hh * 128:(hh + 1) * 128, :],
                            preferred_element_type=jnp.float32)
    h = acc * 0.25 + b3_ref[...]
    h_ref[...] = h

    z1 = jnp.maximum(
        jnp.dot(h, gw1_ref[...], preferred_element_type=jnp.float32)
        + gb1_ref[...], 0.0)
    z = jnp.dot(z1, gw2_ref[...], preferred_element_type=jnp.float32)
    z = z + gb2_ref[0, 0]

    blk_max = jnp.max(z)
    m_old = m_s[0, 0]
    m_new = jnp.maximum(m_old, blk_max)
    corr = jnp.exp(m_old - m_new)
    p = jnp.exp(z - m_new)
    s_s[0, 0] = s_s[0, 0] * corr + jnp.sum(p)
    g_s[...] = g_s[...] * corr + jnp.sum(p * h, axis=0, keepdims=True)
    m_s[0, 0] = m_new

    @pl.when(i == nblk - 1)
    def _fin():
        g = g_s[...] / s_s[0, 0]
        g = jnp.dot(g, mw_ref[...], preferred_element_type=jnp.float32)
        g_ref[...] = jnp.maximum(g + mb_ref[...], 0.0)


def _tail(a3, dp, w3s, b3, gw1, gb1, gw2, gb2, mw, mb, n, blk=400):
    hid = w3s.shape[1]
    return pl.pallas_call(
        _tail_body,
        grid=(n // blk,),
        in_specs=[
            pl.BlockSpec((4, blk, 128), lambda i: (0, i, 0)),
            pl.BlockSpec((2, blk, 16), lambda i: (0, i, 0)),
            pl.BlockSpec((512, hid), lambda i: (0, 0)),
            pl.BlockSpec((1, hid), lambda i: (0, 0)),
            pl.BlockSpec((hid, hid), lambda i: (0, 0)),
            pl.BlockSpec((1, hid), lambda i: (0, 0)),
            pl.BlockSpec((hid, 1), lambda i: (0, 0)),
            pl.BlockSpec((1, 1), lambda i: (0, 0)),
            pl.BlockSpec((hid, hid), lambda i: (0, 0)),
            pl.BlockSpec((1, hid), lambda i: (0, 0)),
        ],
        out_specs=[
            pl.BlockSpec((blk, hid), lambda i: (i, 0)),
            pl.BlockSpec((1, hid), lambda i: (0, 0)),
        ],
        out_shape=[
            jax.ShapeDtypeStruct((n, hid), jnp.float32),
            jax.ShapeDtypeStruct((1, hid), jnp.float32),
        ],
        scratch_shapes=[
            pltpu.SMEM((1, 1), jnp.float32),
            pltpu.SMEM((1, 1), jnp.float32),
            pltpu.VMEM((1, hid), jnp.float32),
        ],
    )(a3, dp, w3s, b3, gw1, gb1, gw2, gb2, mw, mb)


def kernel(x, edge_index, W1, b1, W2, b2, W3, att_src, att_dst, b3,
           gate_W1, gate_b1, gate_W2, gate_b2, mlp_W, mlp_b):
    n = x.shape[0]
    heads, hidden = att_src.shape
    in2 = W3.shape[0]
    npad = _NPAD

    loop = jnp.arange(n, dtype=jnp.int32)
    src = jnp.concatenate([edge_index[0], loop])
    dst = jnp.concatenate([edge_index[1], loop])
    e_real = src.shape[0]
    etot = ((e_real + 4095) // 4096) * 4096
    padn = etot - e_real
    srcp = jnp.concatenate([src, jnp.zeros((padn,), jnp.int32)])
    dstp = jnp.concatenate([dst, jnp.full((padn,), n, jnp.int32)])

    zeros16 = jnp.zeros((npad, 16), jnp.float32)
    zeros64 = jnp.zeros((npad, 64), jnp.float32)
    ones16 = jnp.ones((_CH, 16), jnp.float32)

    degp = _deg_sc(dstp, ones16, zeros16, etot)
    deg = degp[0, :, 0] + degp[1, :, 0]
    dinv = lax.rsqrt(jnp.maximum(deg, 1e-12))
    dinv2 = dinv[:, None]

    xs16 = jnp.zeros((npad, 16), jnp.float32)
    xs16 = xs16.at[:n, :3].set(x * dinv2[:n])
    agg1 = _segsum_sc(xs16, srcp, dstp, zeros16, 16, etot)
    w1p = jnp.zeros((16, 64), jnp.float32).at[:3].set(W1)
    x1s = _gcn_dense(agg1, dinv2, w1p, b1.reshape(1, 64))

    agg2 = _segsum_sc(x1s, srcp, dstp, zeros64, 64, etot)
    w3r = W3.reshape(in2, heads, hidden)
    avs = jnp.einsum("khj,hj->kh", w3r, att_src)
    avd = jnp.einsum("khj,hj->kh", w3r, att_dst)
    x2, asT, adT, cT = _gat_prep(agg2, dinv2, W2, b2.reshape(1, in2),
                                 avs, avd)

    exT, dpart = _gat_edge_sc(asT, adT, cT, srcp, dstp, zeros16, etot)
    zeros128 = jnp.zeros((npad, 128), jnp.float32)
    xq = [x2[:, 32 * q:32 * (q + 1)] for q in range(4)]
    aq = _gat_agg_sc(xq[0], xq[1], xq[2], xq[3], exT, srcp, dstp,
                     zeros128, etot)
    a3 = aq.reshape(4, npad, 4, 32).transpose(2, 1, 0, 3).reshape(
        4, npad, 128)

    w3s = w3r.transpose(1, 0, 2).reshape(heads * in2, hidden)
    h, g = _tail(a3, dpart, w3s, b3.reshape(1, hidden),
                 gate_W1, gate_b1.reshape(1, hidden),
                 gate_W2, gate_b2.reshape(1, 1),
                 mlp_W, mlp_b.reshape(1, hidden), n)
    return (g, h)


# speedup vs baseline: 21.6445x; 1.1868x over previous
import functools

import jax
import jax.numpy as jnp
from jax import lax
from jax.experimental import pallas as pl
from jax.experimental.pallas import tpu as pltpu
from jax.experimental.pallas import tpu_sc as plsc

_NEG_INF = float("-inf")
_NPAD = 10240
_CH = 128


def _segsum_sc(vals, srci, dsti, zeros, f, etot):
    npad = _NPAD
    cpt = etot // (32 * _CH)
    ept = etot // 32
    rpt = npad // 16
    mesh = plsc.VectorSubcoreMesh(core_axis_name="c", subcore_axis_name="s")

    @functools.partial(
        pl.kernel,
        out_type=jax.ShapeDtypeStruct((2, npad, f), jnp.float32),
        mesh=mesh,
        scratch_types=[
            pltpu.VMEM((_CH,), jnp.int32),
            pltpu.VMEM((_CH,), jnp.int32),
            pltpu.VMEM((_CH,), jnp.int32),
            pltpu.VMEM((_CH,), jnp.int32),
            pltpu.VMEM((_CH, f), jnp.float32),
            pltpu.VMEM((_CH, f), jnp.float32),
            pltpu.VMEM_SHARED((npad, f), jnp.float32),
            pltpu.SemaphoreType.DMA,
            pltpu.SemaphoreType.DMA,
        ],
        compiler_params=pltpu.CompilerParams(use_tc_tiling_on_sc=False),
    )
    def k(vals_h, srci_h, dsti_h, zeros_h, out_h,
          sidx0, sidx1, didx0, didx1, rows0, rows1, acc, sem0, sem1):
        cid = lax.axis_index("c")
        sid = lax.axis_index("s")
        wid = sid * 2 + cid
        sidxs = (sidx0, sidx1)
        didxs = (didx0, didx1)
        rows = (rows0, rows1)
        sems = (sem0, sem1)
        pltpu.sync_copy(zeros_h.at[pl.ds(sid * rpt, rpt)],
                        acc.at[pl.ds(sid * rpt, rpt)])
        plsc.subcore_barrier()

        def load_issue(j, b):
            base = wid * ept + j * _CH
            pltpu.sync_copy(srci_h.at[pl.ds(base, _CH)], sidxs[b])
            pltpu.sync_copy(dsti_h.at[pl.ds(base, _CH)], didxs[b])
            pltpu.async_copy(vals_h.at[sidxs[b]], rows[b], sems[b])

        load_issue(0, 0)
        load_issue(1, 1)

        def body(j2, carry):
            for b in range(2):
                j = j2 * 2 + b
                pltpu.make_async_copy(vals_h.at[sidxs[b]], rows[b],
                                      sems[b]).wait()
                pltpu.sync_copy(rows[b], acc.at[didxs[b]], add=True)

                @pl.when(j + 2 < cpt)
                def _next():
                    load_issue(j + 2, b)
            return carry

        lax.fori_loop(0, cpt // 2, body, 0)
        plsc.subcore_barrier()
        pltpu.sync_copy(acc.at[pl.ds(sid * rpt, rpt)],
                        out_h.at[cid, pl.ds(sid * rpt, rpt)])

    return k(vals, srci, dsti, zeros)


def _deg_sc(dsti, ones_rows, zeros, etot):
    npad = _NPAD
    cpt = etot // (32 * _CH)
    ept = etot // 32
    rpt = npad // 16
    mesh = plsc.VectorSubcoreMesh(core_axis_name="c", subcore_axis_name="s")

    @functools.partial(
        pl.kernel,
        out_type=jax.ShapeDtypeStruct((2, npad, 16), jnp.float32),
        mesh=mesh,
        scratch_types=[
            pltpu.VMEM((_CH,), jnp.int32),
            pltpu.VMEM((_CH, 16), jnp.float32),
            pltpu.VMEM_SHARED((npad, 16), jnp.float32),
        ],
        compiler_params=pltpu.CompilerParams(use_tc_tiling_on_sc=False),
    )
    def k(dsti_h, ones_h, zeros_h, out_h, didx, rows, acc):
        cid = lax.axis_index("c")
        sid = lax.axis_index("s")
        wid = sid * 2 + cid
        pltpu.sync_copy(ones_h, rows)
        pltpu.sync_copy(zeros_h.at[pl.ds(sid * rpt, rpt)],
                        acc.at[pl.ds(sid * rpt, rpt)])
        plsc.subcore_barrier()

        def body(j, carry):
            base = wid * ept + j * _CH
            pltpu.sync_copy(dsti_h.at[pl.ds(base, _CH)], didx)
            pltpu.sync_copy(rows, acc.at[didx], add=True)
            return carry

        lax.fori_loop(0, cpt, body, 0)
        plsc.subcore_barrier()
        pltpu.sync_copy(acc.at[pl.ds(sid * rpt, rpt)],
                        out_h.at[cid, pl.ds(sid * rpt, rpt)])

    return k(dsti, ones_rows, zeros)


def _gat_edge_sc(asT, adT, cT, srci, dsti, zeros16, etot):
    npad = _NPAD
    cpt = etot // (32 * _CH)
    ept = etot // 32
    rpt = npad // 16
    mesh = plsc.VectorSubcoreMesh(core_axis_name="c", subcore_axis_name="s")

    @functools.partial(
        pl.kernel,
        out_type=[
            jax.ShapeDtypeStruct((etot, 16), jnp.float32),
            jax.ShapeDtypeStruct((2, npad, 16), jnp.float32),
        ],
        mesh=mesh,
        scratch_types=[
            pltpu.VMEM((_CH,), jnp.int32),
            pltpu.VMEM((_CH,), jnp.int32),
            pltpu.VMEM((_CH, 16), jnp.float32),
            pltpu.VMEM((_CH, 16), jnp.float32),
            pltpu.VMEM((_CH, 16), jnp.float32),
            pltpu.VMEM((_CH, 16), jnp.float32),
            pltpu.VMEM_SHARED((npad, 16), jnp.float32),
            pltpu.SemaphoreType.DMA,
            pltpu.SemaphoreType.DMA,
            pltpu.SemaphoreType.DMA,
        ],
        compiler_params=pltpu.CompilerParams(use_tc_tiling_on_sc=False),
    )
    def k(as_h, ad_h, c_h, srci_h, dsti_h, zeros_h, ex_h, dn_h,
          sidx, didx, ar, br, cr, exb, acc, sem, sem2, sem3):
        cid = lax.axis_index("c")
        sid = lax.axis_index("s")
        wid = sid * 2 + cid
        pltpu.sync_copy(zeros_h.at[pl.ds(sid * rpt, rpt)],
                        acc.at[pl.ds(sid * rpt, rpt)])
        plsc.subcore_barrier()
        mask = lax.iota(jnp.int32, 16) < 4

        def body(j, carry):
            base = wid * ept + j * _CH
            pltpu.sync_copy(srci_h.at[pl.ds(base, _CH)], sidx)
            pltpu.sync_copy(dsti_h.at[pl.ds(base, _CH)], didx)
            pltpu.async_copy(as_h.at[sidx], ar, sem)
            pltpu.async_copy(ad_h.at[didx], br, sem2)
            pltpu.async_copy(c_h.at[didx], cr, sem3)
            pltpu.make_async_copy(as_h.at[sidx], ar, sem).wait()
            pltpu.make_async_copy(ad_h.at[didx], br, sem2).wait()
            pltpu.make_async_copy(c_h.at[didx], cr, sem3).wait()

            @plsc.parallel_loop(0, _CH, 1, unroll=4)
            def inner(e):
                pre = ar[e, :] + br[e, :]
                alpha = jnp.maximum(pre, 0.2 * pre)
                exv = jnp.exp(alpha - cr[e, :])
                exb[e, :] = jnp.where(mask, exv, 0.0)
            pltpu.sync_copy(exb, ex_h.at[pl.ds(base, _CH)])
            pltpu.sync_copy(exb, acc.at[didx], add=True)
            return carry

        lax.fori_loop(0, cpt, body, 0)
        plsc.subcore_barrier()
        pltpu.sync_copy(acc.at[pl.ds(sid * rpt, rpt)],
                        dn_h.at[cid, pl.ds(sid * rpt, rpt)])

    return k(asT, adT, cT, srci, dsti, zeros16)


def _gat_agg_sc(xq0, xq1, xq2, xq3, ex, srci, dsti, zeros128, etot):
    npad = _NPAD
    ept = etot // 16
    cpt = ept // _CH
    rpt = npad // 16
    mesh = plsc.VectorSubcoreMesh(core_axis_name="c", subcore_axis_name="s")

    @functools.partial(
        pl.kernel,
        out_type=jax.ShapeDtypeStruct((2, 2, npad, 128), jnp.float32),
        mesh=mesh,
        scratch_types=[
            pltpu.VMEM((_CH,), jnp.int32),
            pltpu.VMEM((_CH,), jnp.int32),
            pltpu.VMEM((_CH,), jnp.int32),
            pltpu.VMEM((_CH,), jnp.int32),
            pltpu.VMEM((_CH, 32), jnp.float32),
            pltpu.VMEM((_CH, 32), jnp.float32),
            pltpu.VMEM((_CH, 16), jnp.float32),
            pltpu.VMEM((_CH, 128), jnp.float32),
            pltpu.VMEM_SHARED((npad, 128), jnp.float32),
            pltpu.SemaphoreType.DMA,
            pltpu.SemaphoreType.DMA,
        ],
        compiler_params=pltpu.CompilerParams(use_tc_tiling_on_sc=False),
    )
    def k(x0_h, x1_h, x2_h, x3_h, ex_h, srci_h, dsti_h, zeros_h, out_h,
          sidx0, sidx1, didx0, didx1, xg0, xg1, exg, scb, acc, sem0, sem1):
        cid = lax.axis_index("c")
        sid = lax.axis_index("s")
        sidxs = (sidx0, sidx1)
        didxs = (didx0, didx1)
        xgs = (xg0, xg1)
        sems = (sem0, sem1)

        for p in range(2):
            xq_h = (x0_h, x2_h) if p == 0 else (x1_h, x3_h)
            pltpu.sync_copy(zeros_h.at[pl.ds(sid * rpt, rpt)],
                            acc.at[pl.ds(sid * rpt, rpt)])
            plsc.subcore_barrier()

            def load_idx(j, b):
                base = sid * ept + j * _CH
                pltpu.sync_copy(srci_h.at[pl.ds(base, _CH)], sidxs[b])
                pltpu.sync_copy(dsti_h.at[pl.ds(base, _CH)], didxs[b])

            def issue(b, q0_h, q1_h):
                @pl.when(cid == 0)
                def _a():
                    pltpu.async_copy(q0_h.at[sidxs[b]], xgs[b], sems[b])

                @pl.when(cid == 1)
                def _b():
                    pltpu.async_copy(q1_h.at[sidxs[b]], xgs[b], sems[b])

            def wait(b, q0_h, q1_h):
                @pl.when(cid == 0)
                def _a():
                    pltpu.make_async_copy(q0_h.at[sidxs[b]], xgs[b],
                                          sems[b]).wait()

                @pl.when(cid == 1)
                def _b():
                    pltpu.make_async_copy(q1_h.at[sidxs[b]], xgs[b],
                                          sems[b]).wait()

            q0_h, q1_h = xq_h
            load_idx(0, 0)
            issue(0, q0_h, q1_h)
            load_idx(1, 1)
            issue(1, q0_h, q1_h)

            def body(j2, carry):
                for b in range(2):
                    j = j2 * 2 + b
                    base = sid * ept + j * _CH
                    wait(b, q0_h, q1_h)
                    pltpu.sync_copy(ex_h.at[pl.ds(base, _CH)], exg)
                    xg = xgs[b]

                    @plsc.parallel_loop(0, _CH, 1, unroll=4)
                    def inner(e):
                        exv = exg[e, :]
                        v0 = xg[e, pl.ds(0, 16)]
                        v1 = xg[e, pl.ds(16, 16)]
                        for hh in range(4):
                            w = exv[hh]
                            scb[e, pl.ds(hh * 32, 16)] = v0 * w
                            scb[e, pl.ds(hh * 32 + 16, 16)] = v1 * w
                    pltpu.sync_copy(scb, acc.at[didxs[b]], add=True)

                    @pl.when(j + 2 < cpt)
                    def _next():
                        load_idx(j + 2, b)
                        issue(b, q0_h, q1_h)
                return carry

            lax.fori_loop(0, cpt // 2, body, 0)
            plsc.subcore_barrier()
            pltpu.sync_copy(acc.at[pl.ds(sid * rpt, rpt)],
                            out_h.at[cid, p, pl.ds(sid * rpt, rpt)])
            plsc.subcore_barrier()

    return k(xq0, xq1, xq2, xq3, ex, srci, dsti, zeros128)


def _gcn_dense_body(p_ref, dinv_ref, w_ref, b_ref, out_ref):
    s = (p_ref[0] + p_ref[1]) * dinv_ref[...]
    h = jnp.dot(s, w_ref[...], preferred_element_type=jnp.float32)
    h = jnp.maximum(h + b_ref[...], 0.0)
    out_ref[...] = h * dinv_ref[...]


def _gcn_dense(p, dinv, w, b, blk=1280):
    npad, fin = p.shape[1], p.shape[2]
    fout = w.shape[1]
    return pl.pallas_call(
        _gcn_dense_body,
        grid=(npad // blk,),
        in_specs=[
            pl.BlockSpec((2, blk, fin), lambda i: (0, i, 0)),
            pl.BlockSpec((blk, 1), lambda i: (i, 0)),
            pl.BlockSpec((fin, fout), lambda i: (0, 0)),
            pl.BlockSpec((1, fout), lambda i: (0, 0)),
        ],
        out_specs=pl.BlockSpec((blk, fout), lambda i: (i, 0)),
        out_shape=jax.ShapeDtypeStruct((npad, fout), jnp.float32),
    )(p, dinv, w, b)


def _gat_prep_body(p_ref, dinv_ref, w_ref, b_ref, avs_ref, avd_ref,
                   x2_ref, as_ref, ad_ref, c_ref):
    s = (p_ref[0] + p_ref[1]) * dinv_ref[...]
    x2 = jnp.dot(s, w_ref[...], preferred_element_type=jnp.float32)
    x2 = jnp.maximum(x2 + b_ref[...], 0.0)
    x2_ref[...] = x2
    a_s = jnp.dot(x2, avs_ref[...], preferred_element_type=jnp.float32)
    a_d = jnp.dot(x2, avd_ref[...], preferred_element_type=jnp.float32)
    pre = a_s + a_d
    c = jnp.maximum(pre, 0.2 * pre)
    z = jnp.zeros((a_s.shape[0], 12), jnp.float32)
    as_ref[...] = jnp.concatenate([a_s, z], axis=1)
    ad_ref[...] = jnp.concatenate([a_d, z], axis=1)
    c_ref[...] = jnp.concatenate([c, z], axis=1)


def _gat_prep(p, dinv, w, b, avs, avd, blk=1280):
    npad, fin = p.shape[1], p.shape[2]
    fout = w.shape[1]
    return pl.pallas_call(
        _gat_prep_body,
        grid=(npad // blk,),
        in_specs=[
            pl.BlockSpec((2, blk, fin), lambda i: (0, i, 0)),
            pl.BlockSpec((blk, 1), lambda i: (i, 0)),
            pl.BlockSpec((fin, fout), lambda i: (0, 0)),
            pl.BlockSpec((1, fout), lambda i: (0, 0)),
            pl.BlockSpec((fout, 4), lambda i: (0, 0)),
            pl.BlockSpec((fout, 4), lambda i: (0, 0)),
        ],
        out_specs=[
            pl.BlockSpec((blk, fout), lambda i: (i, 0)),
            pl.BlockSpec((blk, 16), lambda i: (i, 0)),
            pl.BlockSpec((blk, 16), lambda i: (i, 0)),
            pl.BlockSpec((blk, 16), lambda i: (i, 0)),
        ],
        out_shape=[
            jax.ShapeDtypeStruct((npad, fout), jnp.float32),
            jax.ShapeDtypeStruct((npad, 16), jnp.float32),
            jax.ShapeDtypeStruct((npad, 16), jnp.float32),
            jax.ShapeDtypeStruct((npad, 16), jnp.float32),
        ],
    )(p, dinv, w, b, avs, avd)


def _tail_body(a3_ref, dp_ref, w3s_ref, b3_ref, gw1_ref, gb1_ref, gw2_ref,
               gb2_ref, mw_ref, mb_ref, h_ref, g_ref, m_s, s_s, g_s):
    i = pl.program_id(0)
    nblk = pl.num_programs(0)

    @pl.when(i == 0)
    def _init():
        m_s[0, 0] = _NEG_INF
        s_s[0, 0] = 0.0
        g_s[...] = jnp.zeros_like(g_s)

    d = dp_ref[0] + dp_ref[1]
    r = 1.0 / (d + 1e-16)
    hid = h_ref.shape[1]
    acc = jnp.zeros((a3_ref.shape[1], hid), jnp.float32)
    for hh in range(4):
        sc = a3_ref[hh] * r[:, hh:hh + 1]
        acc = acc + jnp.dot(sc, w3s_ref[hh * 128:(hh + 1) * 128, :],
                            preferred_element_type=jnp.float32)
    h = acc * 0.25 + b3_ref[...]
    h_ref[...] = h

    z1 = jnp.maximum(
        jnp.dot(h, gw1_ref[...], preferred_element_type=jnp.float32)
        + gb1_ref[...], 0.0)
    z = jnp.dot(z1, gw2_ref[...], preferred_element_type=jnp.float32)
    z = z + gb2_ref[0, 0]

    blk_max = jnp.max(z)
    m_old = m_s[0, 0]
    m_new = jnp.maximum(m_old, blk_max)
    corr = jnp.exp(m_old - m_new)
    p = jnp.exp(z - m_new)
    s_s[0, 0] = s_s[0, 0] * corr + jnp.sum(p)
    g_s[...] = g_s[...] * corr + jnp.sum(p * h, axis=0, keepdims=True)
    m_s[0, 0] = m_new

    @pl.when(i == nblk - 1)
    def _fin():
        g = g_s[...] / s_s[0, 0]
        g = jnp.dot(g, mw_ref[...], preferred_element_type=jnp.float32)
        g_ref[...] = jnp.maximum(g + mb_ref[...], 0.0)


def _tail(a3, dp, w3s, b3, gw1, gb1, gw2, gb2, mw, mb, n, blk=400):
    hid = w3s.shape[1]
    return pl.pallas_call(
        _tail_body,
        grid=(n // blk,),
        in_specs=[
            pl.BlockSpec((4, blk, 128), lambda i: (0, i, 0)),
            pl.BlockSpec((2, blk, 16), lambda i: (0, i, 0)),
            pl.BlockSpec((512, hid), lambda i: (0, 0)),
            pl.BlockSpec((1, hid), lambda i: (0, 0)),
            pl.BlockSpec((hid, hid), lambda i: (0, 0)),
            pl.BlockSpec((1, hid), lambda i: (0, 0)),
            pl.BlockSpec((hid, 1), lambda i: (0, 0)),
            pl.BlockSpec((1, 1), lambda i: (0, 0)),
            pl.BlockSpec((hid, hid), lambda i: (0, 0)),
            pl.BlockSpec((1, hid), lambda i: (0, 0)),
        ],
        out_specs=[
            pl.BlockSpec((blk, hid), lambda i: (i, 0)),
            pl.BlockSpec((1, hid), lambda i: (0, 0)),
        ],
        out_shape=[
            jax.ShapeDtypeStruct((n, hid), jnp.float32),
            jax.ShapeDtypeStruct((1, hid), jnp.float32),
        ],
        scratch_shapes=[
            pltpu.SMEM((1, 1), jnp.float32),
            pltpu.SMEM((1, 1), jnp.float32),
            pltpu.VMEM((1, hid), jnp.float32),
        ],
    )(a3, dp, w3s, b3, gw1, gb1, gw2, gb2, mw, mb)


def kernel(x, edge_index, W1, b1, W2, b2, W3, att_src, att_dst, b3,
           gate_W1, gate_b1, gate_W2, gate_b2, mlp_W, mlp_b):
    n = x.shape[0]
    heads, hidden = att_src.shape
    in2 = W3.shape[0]
    npad = _NPAD

    loop = jnp.arange(n, dtype=jnp.int32)
    src = jnp.concatenate([edge_index[0], loop])
    dst = jnp.concatenate([edge_index[1], loop])
    e_real = src.shape[0]
    etot = ((e_real + 4095) // 4096) * 4096
    padn = etot - e_real
    srcp = jnp.concatenate([src, jnp.zeros((padn,), jnp.int32)])
    dstp = jnp.concatenate([dst, jnp.full((padn,), n, jnp.int32)])

    zeros16 = jnp.zeros((npad, 16), jnp.float32)
    zeros64 = jnp.zeros((npad, 64), jnp.float32)
    ones16 = jnp.ones((_CH, 16), jnp.float32)

    degp = _deg_sc(dstp, ones16, zeros16, etot)
    deg = degp[0, :, 0] + degp[1, :, 0]
    dinv = lax.rsqrt(jnp.maximum(deg, 1e-12))
    dinv2 = dinv[:, None]

    xs16 = jnp.zeros((npad, 16), jnp.float32)
    xs16 = xs16.at[:n, :3].set(x * dinv2[:n])
    agg1 = _segsum_sc(xs16, srcp, dstp, zeros16, 16, etot)
    w1p = jnp.zeros((16, 64), jnp.float32).at[:3].set(W1)
    x1s = _gcn_dense(agg1, dinv2, w1p, b1.reshape(1, 64))

    agg2 = _segsum_sc(x1s, srcp, dstp, zeros64, 64, etot)
    w3r = W3.reshape(in2, heads, hidden)
    avs = jnp.einsum("khj,hj->kh", w3r, att_src)
    avd = jnp.einsum("khj,hj->kh", w3r, att_dst)
    x2, asT, adT, cT = _gat_prep(agg2, dinv2, W2, b2.reshape(1, in2),
                                 avs, avd)

    exT, dpart = _gat_edge_sc(asT, adT, cT, srcp, dstp, zeros16, etot)
    zeros128 = jnp.zeros((npad, 128), jnp.float32)
    xq = [x2[:, 32 * q:32 * (q + 1)] for q in range(4)]
    aq = _gat_agg_sc(xq[0], xq[1], xq[2], xq[3], exT, srcp, dstp,
                     zeros128, etot)
    a3 = aq.reshape(4, npad, 4, 32).transpose(2, 1, 0, 3).reshape(
        4, npad, 128)

    w3s = w3r.transpose(1, 0, 2).reshape(heads * in2, hidden)
    h, g = _tail(a3, dpart, w3s, b3.reshape(1, hidden),
                 gate_W1, gate_b1.reshape(1, hidden),
                 gate_W2, gate_b2.reshape(1, 1),
                 mlp_W, mlp_b.reshape(1, hidden), n)
    return (g, h)


# speedup vs baseline: 24.0403x; 1.1107x over previous
import functools

import jax
import jax.numpy as jnp
from jax import lax
from jax.experimental import pallas as pl
from jax.experimental.pallas import tpu as pltpu
from jax.experimental.pallas import tpu_sc as plsc

_NEG_INF = float("-inf")
_NPAD = 10240
_CH = 128


def _segsum_sc(vals, srci, dsti, zeros, f, etot):
    npad = _NPAD
    cpt = etot // (32 * _CH)
    ept = etot // 32
    rpt = npad // 16
    mesh = plsc.VectorSubcoreMesh(core_axis_name="c", subcore_axis_name="s")

    @functools.partial(
        pl.kernel,
        out_type=jax.ShapeDtypeStruct((2, npad, f), jnp.float32),
        mesh=mesh,
        scratch_types=[
            pltpu.VMEM((_CH,), jnp.int32),
            pltpu.VMEM((_CH,), jnp.int32),
            pltpu.VMEM((_CH,), jnp.int32),
            pltpu.VMEM((_CH,), jnp.int32),
            pltpu.VMEM((_CH, f), jnp.float32),
            pltpu.VMEM((_CH, f), jnp.float32),
            pltpu.VMEM_SHARED((npad, f), jnp.float32),
            pltpu.SemaphoreType.DMA,
            pltpu.SemaphoreType.DMA,
        ],
        compiler_params=pltpu.CompilerParams(use_tc_tiling_on_sc=False),
    )
    def k(vals_h, srci_h, dsti_h, zeros_h, out_h,
          sidx0, sidx1, didx0, didx1, rows0, rows1, acc, sem0, sem1):
        cid = lax.axis_index("c")
        sid = lax.axis_index("s")
        wid = sid * 2 + cid
        sidxs = (sidx0, sidx1)
        didxs = (didx0, didx1)
        rows = (rows0, rows1)
        sems = (sem0, sem1)
        pltpu.sync_copy(zeros_h.at[pl.ds(sid * rpt, rpt)],
                        acc.at[pl.ds(sid * rpt, rpt)])
        plsc.subcore_barrier()

        def load_issue(j, b):
            base = wid * ept + j * _CH
            pltpu.sync_copy(srci_h.at[pl.ds(base, _CH)], sidxs[b])
            pltpu.sync_copy(dsti_h.at[pl.ds(base, _CH)], didxs[b])
            pltpu.async_copy(vals_h.at[sidxs[b]], rows[b], sems[b])

        load_issue(0, 0)
        load_issue(1, 1)

        def body(j2, carry):
            for b in range(2):
                j = j2 * 2 + b
                pltpu.make_async_copy(vals_h.at[sidxs[b]], rows[b],
                                      sems[b]).wait()
                pltpu.sync_copy(rows[b], acc.at[didxs[b]], add=True)

                @pl.when(j + 2 < cpt)
                def _next():
                    load_issue(j + 2, b)
            return carry

        lax.fori_loop(0, cpt // 2, body, 0)
        plsc.subcore_barrier()
        pltpu.sync_copy(acc.at[pl.ds(sid * rpt, rpt)],
                        out_h.at[cid, pl.ds(sid * rpt, rpt)])

    return k(vals, srci, dsti, zeros)


def _deg_sc(dsti, ones_rows, zeros, etot):
    npad = _NPAD
    cpt = etot // (32 * _CH)
    ept = etot // 32
    rpt = npad // 16
    mesh = plsc.VectorSubcoreMesh(core_axis_name="c", subcore_axis_name="s")

    @functools.partial(
        pl.kernel,
        out_type=jax.ShapeDtypeStruct((2, npad, 16), jnp.float32),
        mesh=mesh,
        scratch_types=[
            pltpu.VMEM((_CH,), jnp.int32),
            pltpu.VMEM((_CH, 16), jnp.float32),
            pltpu.VMEM_SHARED((npad, 16), jnp.float32),
        ],
        compiler_params=pltpu.CompilerParams(use_tc_tiling_on_sc=False),
    )
    def k(dsti_h, ones_h, zeros_h, out_h, didx, rows, acc):
        cid = lax.axis_index("c")
        sid = lax.axis_index("s")
        wid = sid * 2 + cid
        pltpu.sync_copy(ones_h, rows)
        pltpu.sync_copy(zeros_h.at[pl.ds(sid * rpt, rpt)],
                        acc.at[pl.ds(sid * rpt, rpt)])
        plsc.subcore_barrier()

        def body(j, carry):
            base = wid * ept + j * _CH
            pltpu.sync_copy(dsti_h.at[pl.ds(base, _CH)], didx)
            pltpu.sync_copy(rows, acc.at[didx], add=True)
            return carry

        lax.fori_loop(0, cpt, body, 0)
        plsc.subcore_barrier()
        pltpu.sync_copy(acc.at[pl.ds(sid * rpt, rpt)],
                        out_h.at[cid, pl.ds(sid * rpt, rpt)])

    return k(dsti, ones_rows, zeros)


def _gat_edge_sc(asT, adT, cT, srci, dsti, zeros16, etot):
    npad = _NPAD
    cpt = etot // (32 * _CH)
    ept = etot // 32
    rpt = npad // 16
    mesh = plsc.VectorSubcoreMesh(core_axis_name="c", subcore_axis_name="s")

    @functools.partial(
        pl.kernel,
        out_type=[
            jax.ShapeDtypeStruct((etot, 16), jnp.float32),
            jax.ShapeDtypeStruct((2, npad, 16), jnp.float32),
        ],
        mesh=mesh,
        scratch_types=[
            pltpu.VMEM((_CH,), jnp.int32),
            pltpu.VMEM((_CH,), jnp.int32),
            pltpu.VMEM((_CH, 16), jnp.float32),
            pltpu.VMEM((_CH, 16), jnp.float32),
            pltpu.VMEM((_CH, 16), jnp.float32),
            pltpu.VMEM((_CH, 16), jnp.float32),
            pltpu.VMEM_SHARED((npad, 16), jnp.float32),
            pltpu.SemaphoreType.DMA,
            pltpu.SemaphoreType.DMA,
            pltpu.SemaphoreType.DMA,
        ],
        compiler_params=pltpu.CompilerParams(use_tc_tiling_on_sc=False),
    )
    def k(as_h, ad_h, c_h, srci_h, dsti_h, zeros_h, ex_h, dn_h,
          sidx, didx, ar, br, cr, exb, acc, sem, sem2, sem3):
        cid = lax.axis_index("c")
        sid = lax.axis_index("s")
        wid = sid * 2 + cid
        pltpu.sync_copy(zeros_h.at[pl.ds(sid * rpt, rpt)],
                        acc.at[pl.ds(sid * rpt, rpt)])
        plsc.subcore_barrier()
        mask = lax.iota(jnp.int32, 16) < 4

        def body(j, carry):
            base = wid * ept + j * _CH
            pltpu.sync_copy(srci_h.at[pl.ds(base, _CH)], sidx)
            pltpu.sync_copy(dsti_h.at[pl.ds(base, _CH)], didx)
            pltpu.async_copy(as_h.at[sidx], ar, sem)
            pltpu.async_copy(ad_h.at[didx], br, sem2)
            pltpu.async_copy(c_h.at[didx], cr, sem3)
            pltpu.make_async_copy(as_h.at[sidx], ar, sem).wait()
            pltpu.make_async_copy(ad_h.at[didx], br, sem2).wait()
            pltpu.make_async_copy(c_h.at[didx], cr, sem3).wait()

            @plsc.parallel_loop(0, _CH, 1, unroll=4)
            def inner(e):
                pre = ar[e, :] + br[e, :]
                alpha = jnp.maximum(pre, 0.2 * pre)
                exv = jnp.exp(alpha - cr[e, :])
                exb[e, :] = jnp.where(mask, exv, 0.0)
            pltpu.sync_copy(exb, ex_h.at[pl.ds(base, _CH)])
            pltpu.sync_copy(exb, acc.at[didx], add=True)
            return carry

        lax.fori_loop(0, cpt, body, 0)
        plsc.subcore_barrier()
        pltpu.sync_copy(acc.at[pl.ds(sid * rpt, rpt)],
                        dn_h.at[cid, pl.ds(sid * rpt, rpt)])

    return k(asT, adT, cT, srci, dsti, zeros16)


def _gat_agg_sc(xq0, xq1, xq2, xq3, ex, srci, dsti, zeros128, etot):
    npad = _NPAD
    ept = etot // 16
    cpt = ept // _CH
    rpt = npad // 16
    mesh = plsc.VectorSubcoreMesh(core_axis_name="c", subcore_axis_name="s")

    @functools.partial(
        pl.kernel,
        out_type=jax.ShapeDtypeStruct((2, 2, npad, 128), jnp.float32),
        mesh=mesh,
        scratch_types=[
            pltpu.VMEM((_CH,), jnp.int32),
            pltpu.VMEM((_CH,), jnp.int32),
            pltpu.VMEM((_CH,), jnp.int32),
            pltpu.VMEM((_CH,), jnp.int32),
            pltpu.VMEM((_CH, 32), jnp.float32),
            pltpu.VMEM((_CH, 32), jnp.float32),
            pltpu.VMEM((_CH, 16), jnp.float32),
            pltpu.VMEM((_CH, 128), jnp.float32),
            pltpu.VMEM((_CH, 128), jnp.float32),
            pltpu.VMEM_SHARED((npad, 128), jnp.float32),
            pltpu.SemaphoreType.DMA,
            pltpu.SemaphoreType.DMA,
            pltpu.SemaphoreType.DMA,
            pltpu.SemaphoreType.DMA,
        ],
        compiler_params=pltpu.CompilerParams(use_tc_tiling_on_sc=False),
    )
    def k(x0_h, x1_h, x2_h, x3_h, ex_h, srci_h, dsti_h, zeros_h, out_h,
          sidx0, sidx1, didx0, didx1, xg0, xg1, exg, scb0, scb1, acc,
          sem0, sem1, ssem0, ssem1):
        cid = lax.axis_index("c")
        sid = lax.axis_index("s")
        sidxs = (sidx0, sidx1)
        didxs = (didx0, didx1)
        xgs = (xg0, xg1)
        scbs = (scb0, scb1)
        sems = (sem0, sem1)
        ssems = (ssem0, ssem1)

        for p in range(2):
            xq_h = (x0_h, x2_h) if p == 0 else (x1_h, x3_h)
            pltpu.sync_copy(zeros_h.at[pl.ds(sid * rpt, rpt)],
                            acc.at[pl.ds(sid * rpt, rpt)])
            plsc.subcore_barrier()

            def load_sidx(j, b):
                base = sid * ept + j * _CH
                pltpu.sync_copy(srci_h.at[pl.ds(base, _CH)], sidxs[b])

            def load_didx(j, b):
                base = sid * ept + j * _CH
                pltpu.sync_copy(dsti_h.at[pl.ds(base, _CH)], didxs[b])

            def issue(b, q0_h, q1_h):
                @pl.when(cid == 0)
                def _a():
                    pltpu.async_copy(q0_h.at[sidxs[b]], xgs[b], sems[b])

                @pl.when(cid == 1)
                def _b():
                    pltpu.async_copy(q1_h.at[sidxs[b]], xgs[b], sems[b])

            def wait(b, q0_h, q1_h):
                @pl.when(cid == 0)
                def _a():
                    pltpu.make_async_copy(q0_h.at[sidxs[b]], xgs[b],
                                          sems[b]).wait()

                @pl.when(cid == 1)
                def _b():
                    pltpu.make_async_copy(q1_h.at[sidxs[b]], xgs[b],
                                          sems[b]).wait()

            q0_h, q1_h = xq_h
            load_sidx(0, 0)
            issue(0, q0_h, q1_h)
            load_sidx(1, 1)
            issue(1, q0_h, q1_h)

            def body(j2, carry):
                for b in range(2):
                    j = j2 * 2 + b
                    base = sid * ept + j * _CH
                    wait(b, q0_h, q1_h)
                    pltpu.sync_copy(ex_h.at[pl.ds(base, _CH)], exg)
                    xg = xgs[b]
                    scb = scbs[b]

                    @pl.when(j >= 2)
                    def _drain():
                        pltpu.make_async_copy(scb, acc.at[didxs[b]],
                                              ssems[b]).wait()

                    @plsc.parallel_loop(0, _CH, 1, unroll=4)
                    def inner(e):
                        exv = exg[e, :]
                        v0 = xg[e, pl.ds(0, 16)]
                        v1 = xg[e, pl.ds(16, 16)]
                        for hh in range(4):
                            w = exv[hh]
                            scb[e, pl.ds(hh * 32, 16)] = v0 * w
                            scb[e, pl.ds(hh * 32 + 16, 16)] = v1 * w

                    load_didx(j, b)
                    pltpu.async_copy(scb, acc.at[didxs[b]], ssems[b],
                                     add=True)

                    @pl.when(j + 2 < cpt)
                    def _next():
                        load_sidx(j + 2, b)
                        issue(b, q0_h, q1_h)
                return carry

            lax.fori_loop(0, cpt // 2, body, 0)
            for b in range(2):
                pltpu.make_async_copy(scbs[b], acc.at[didxs[b]],
                                      ssems[b]).wait()
            plsc.subcore_barrier()
            pltpu.sync_copy(acc.at[pl.ds(sid * rpt, rpt)],
                            out_h.at[cid, p, pl.ds(sid * rpt, rpt)])
            plsc.subcore_barrier()

    return k(xq0, xq1, xq2, xq3, ex, srci, dsti, zeros128)


def _gcn_dense_body(p_ref, dinv_ref, w_ref, b_ref, out_ref):
    s = (p_ref[0] + p_ref[1]) * dinv_ref[...]
    h = jnp.dot(s, w_ref[...], preferred_element_type=jnp.float32)
    h = jnp.maximum(h + b_ref[...], 0.0)
    out_ref[...] = h * dinv_ref[...]


def _gcn_dense(p, dinv, w, b, blk=1280):
    npad, fin = p.shape[1], p.shape[2]
    fout = w.shape[1]
    return pl.pallas_call(
        _gcn_dense_body,
        grid=(npad // blk,),
        in_specs=[
            pl.BlockSpec((2, blk, fin), lambda i: (0, i, 0)),
            pl.BlockSpec((blk, 1), lambda i: (i, 0)),
            pl.BlockSpec((fin, fout), lambda i: (0, 0)),
            pl.BlockSpec((1, fout), lambda i: (0, 0)),
        ],
        out_specs=pl.BlockSpec((blk, fout), lambda i: (i, 0)),
        out_shape=jax.ShapeDtypeStruct((npad, fout), jnp.float32),
    )(p, dinv, w, b)


def _gat_prep_body(p_ref, dinv_ref, w_ref, b_ref, avs_ref, avd_ref,
                   x2_ref, as_ref, ad_ref, c_ref):
    s = (p_ref[0] + p_ref[1]) * dinv_ref[...]
    x2 = jnp.dot(s, w_ref[...], preferred_element_type=jnp.float32)
    x2 = jnp.maximum(x2 + b_ref[...], 0.0)
    x2_ref[...] = x2
    a_s = jnp.dot(x2, avs_ref[...], preferred_element_type=jnp.float32)
    a_d = jnp.dot(x2, avd_ref[...], preferred_element_type=jnp.float32)
    pre = a_s + a_d
    c = jnp.maximum(pre, 0.2 * pre)
    z = jnp.zeros((a_s.shape[0], 12), jnp.float32)
    as_ref[...] = jnp.concatenate([a_s, z], axis=1)
    ad_ref[...] = jnp.concatenate([a_d, z], axis=1)
    c_ref[...] = jnp.concatenate([c, z], axis=1)


def _gat_prep(p, dinv, w, b, avs, avd, blk=1280):
    npad, fin = p.shape[1], p.shape[2]
    fout = w.shape[1]
    return pl.pallas_call(
        _gat_prep_body,
        grid=(npad // blk,),
        in_specs=[
            pl.BlockSpec((2, blk, fin), lambda i: (0, i, 0)),
            pl.BlockSpec((blk, 1), lambda i: (i, 0)),
            pl.BlockSpec((fin, fout), lambda i: (0, 0)),
            pl.BlockSpec((1, fout), lambda i: (0, 0)),
            pl.BlockSpec((fout, 4), lambda i: (0, 0)),
            pl.BlockSpec((fout, 4), lambda i: (0, 0)),
        ],
        out_specs=[
            pl.BlockSpec((blk, fout), lambda i: (i, 0)),
            pl.BlockSpec((blk, 16), lambda i: (i, 0)),
            pl.BlockSpec((blk, 16), lambda i: (i, 0)),
            pl.BlockSpec((blk, 16), lambda i: (i, 0)),
        ],
        out_shape=[
            jax.ShapeDtypeStruct((npad, fout), jnp.float32),
            jax.ShapeDtypeStruct((npad, 16), jnp.float32),
            jax.ShapeDtypeStruct((npad, 16), jnp.float32),
            jax.ShapeDtypeStruct((npad, 16), jnp.float32),
        ],
    )(p, dinv, w, b, avs, avd)


def _tail_body(a3_ref, dp_ref, w3s_ref, b3_ref, gw1_ref, gb1_ref, gw2_ref,
               gb2_ref, mw_ref, mb_ref, h_ref, g_ref, m_s, s_s, g_s):
    i = pl.program_id(0)
    nblk = pl.num_programs(0)

    @pl.when(i == 0)
    def _init():
        m_s[0, 0] = _NEG_INF
        s_s[0, 0] = 0.0
        g_s[...] = jnp.zeros_like(g_s)

    d = dp_ref[0] + dp_ref[1]
    r = 1.0 / (d + 1e-16)
    hid = h_ref.shape[1]
    acc = jnp.zeros((a3_ref.shape[1], hid), jnp.float32)
    for hh in range(4):
        sc = a3_ref[hh] * r[:, hh:hh + 1]
        acc = acc + jnp.dot(sc, w3s_ref[hh * 128:(hh + 1) * 128, :],
                            preferred_element_type=jnp.float32)
    h = acc * 0.25 + b3_ref[...]
    h_ref[...] = h

    z1 = jnp.maximum(
        jnp.dot(h, gw1_ref[...], preferred_element_type=jnp.float32)
        + gb1_ref[...], 0.0)
    z = jnp.dot(z1, gw2_ref[...], preferred_element_type=jnp.float32)
    z = z + gb2_ref[0, 0]

    blk_max = jnp.max(z)
    m_old = m_s[0, 0]
    m_new = jnp.maximum(m_old, blk_max)
    corr = jnp.exp(m_old - m_new)
    p = jnp.exp(z - m_new)
    s_s[0, 0] = s_s[0, 0] * corr + jnp.sum(p)
    g_s[...] = g_s[...] * corr + jnp.sum(p * h, axis=0, keepdims=True)
    m_s[0, 0] = m_new

    @pl.when(i == nblk - 1)
    def _fin():
        g = g_s[...] / s_s[0, 0]
        g = jnp.dot(g, mw_ref[...], preferred_element_type=jnp.float32)
        g_ref[...] = jnp.maximum(g + mb_ref[...], 0.0)


def _tail(a3, dp, w3s, b3, gw1, gb1, gw2, gb2, mw, mb, n, blk=400):
    hid = w3s.shape[1]
    return pl.pallas_call(
        _tail_body,
        grid=(n // blk,),
        in_specs=[
            pl.BlockSpec((4, blk, 128), lambda i: (0, i, 0)),
            pl.BlockSpec((2, blk, 16), lambda i: (0, i, 0)),
            pl.BlockSpec((512, hid), lambda i: (0, 0)),
            pl.BlockSpec((1, hid), lambda i: (0, 0)),
            pl.BlockSpec((hid, hid), lambda i: (0, 0)),
            pl.BlockSpec((1, hid), lambda i: (0, 0)),
            pl.BlockSpec((hid, 1), lambda i: (0, 0)),
            pl.BlockSpec((1, 1), lambda i: (0, 0)),
            pl.BlockSpec((hid, hid), lambda i: (0, 0)),
            pl.BlockSpec((1, hid), lambda i: (0, 0)),
        ],
        out_specs=[
            pl.BlockSpec((blk, hid), lambda i: (i, 0)),
            pl.BlockSpec((1, hid), lambda i: (0, 0)),
        ],
        out_shape=[
            jax.ShapeDtypeStruct((n, hid), jnp.float32),
            jax.ShapeDtypeStruct((1, hid), jnp.float32),
        ],
        scratch_shapes=[
            pltpu.SMEM((1, 1), jnp.float32),
            pltpu.SMEM((1, 1), jnp.float32),
            pltpu.VMEM((1, hid), jnp.float32),
        ],
    )(a3, dp, w3s, b3, gw1, gb1, gw2, gb2, mw, mb)


def kernel(x, edge_index, W1, b1, W2, b2, W3, att_src, att_dst, b3,
           gate_W1, gate_b1, gate_W2, gate_b2, mlp_W, mlp_b):
    n = x.shape[0]
    heads, hidden = att_src.shape
    in2 = W3.shape[0]
    npad = _NPAD

    loop = jnp.arange(n, dtype=jnp.int32)
    src = jnp.concatenate([edge_index[0], loop])
    dst = jnp.concatenate([edge_index[1], loop])
    e_real = src.shape[0]
    etot = ((e_real + 4095) // 4096) * 4096
    padn = etot - e_real
    srcp = jnp.concatenate([src, jnp.zeros((padn,), jnp.int32)])
    dstp = jnp.concatenate([dst, jnp.full((padn,), n, jnp.int32)])

    zeros16 = jnp.zeros((npad, 16), jnp.float32)
    zeros64 = jnp.zeros((npad, 64), jnp.float32)
    ones16 = jnp.ones((_CH, 16), jnp.float32)

    degp = _deg_sc(dstp, ones16, zeros16, etot)
    deg = degp[0, :, 0] + degp[1, :, 0]
    dinv = lax.rsqrt(jnp.maximum(deg, 1e-12))
    dinv2 = dinv[:, None]

    xs16 = jnp.zeros((npad, 16), jnp.float32)
    xs16 = xs16.at[:n, :3].set(x * dinv2[:n])
    agg1 = _segsum_sc(xs16, srcp, dstp, zeros16, 16, etot)
    w1p = jnp.zeros((16, 64), jnp.float32).at[:3].set(W1)
    x1s = _gcn_dense(agg1, dinv2, w1p, b1.reshape(1, 64))

    agg2 = _segsum_sc(x1s, srcp, dstp, zeros64, 64, etot)
    w3r = W3.reshape(in2, heads, hidden)
    avs = jnp.einsum("khj,hj->kh", w3r, att_src)
    avd = jnp.einsum("khj,hj->kh", w3r, att_dst)
    x2, asT, adT, cT = _gat_prep(agg2, dinv2, W2, b2.reshape(1, in2),
                                 avs, avd)

    exT, dpart = _gat_edge_sc(asT, adT, cT, srcp, dstp, zeros16, etot)
    zeros128 = jnp.zeros((npad, 128), jnp.float32)
    xq = [x2[:, 32 * q:32 * (q + 1)] for q in range(4)]
    aq = _gat_agg_sc(xq[0], xq[1], xq[2], xq[3], exT, srcp, dstp,
                     zeros128, etot)
    a3 = aq.reshape(4, npad, 4, 32).transpose(2, 1, 0, 3).reshape(
        4, npad, 128)

    w3s = w3r.transpose(1, 0, 2).reshape(heads * in2, hidden)
    h, g = _tail(a3, dpart, w3s, b3.reshape(1, hidden),
                 gate_W1, gate_b1.reshape(1, hidden),
                 gate_W2, gate_b2.reshape(1, 1),
                 mlp_W, mlp_b.reshape(1, hidden), n)
    return (g, h)


# speedup vs baseline: 25.7365x; 1.0706x over previous
import functools

import jax
import jax.numpy as jnp
from jax import lax
from jax.experimental import pallas as pl
from jax.experimental.pallas import tpu as pltpu
from jax.experimental.pallas import tpu_sc as plsc

_NEG_INF = float("-inf")
_NPAD = 10240
_CH = 128


def _segsum_sc(vals, srci, dsti, zeros, f, etot):
    npad = _NPAD
    cpt = etot // (32 * _CH)
    ept = etot // 32
    rpt = npad // 16
    mesh = plsc.VectorSubcoreMesh(core_axis_name="c", subcore_axis_name="s")

    @functools.partial(
        pl.kernel,
        out_type=jax.ShapeDtypeStruct((2, npad, f), jnp.float32),
        mesh=mesh,
        scratch_types=[
            pltpu.VMEM((_CH,), jnp.int32),
            pltpu.VMEM((_CH,), jnp.int32),
            pltpu.VMEM((_CH,), jnp.int32),
            pltpu.VMEM((_CH,), jnp.int32),
            pltpu.VMEM((_CH, f), jnp.float32),
            pltpu.VMEM((_CH, f), jnp.float32),
            pltpu.VMEM_SHARED((npad, f), jnp.float32),
            pltpu.SemaphoreType.DMA,
            pltpu.SemaphoreType.DMA,
        ],
        compiler_params=pltpu.CompilerParams(use_tc_tiling_on_sc=False),
    )
    def k(vals_h, srci_h, dsti_h, zeros_h, out_h,
          sidx0, sidx1, didx0, didx1, rows0, rows1, acc, sem0, sem1):
        cid = lax.axis_index("c")
        sid = lax.axis_index("s")
        wid = sid * 2 + cid
        sidxs = (sidx0, sidx1)
        didxs = (didx0, didx1)
        rows = (rows0, rows1)
        sems = (sem0, sem1)
        pltpu.sync_copy(zeros_h.at[pl.ds(sid * rpt, rpt)],
                        acc.at[pl.ds(sid * rpt, rpt)])
        plsc.subcore_barrier()

        def load_issue(j, b):
            base = wid * ept + j * _CH
            pltpu.sync_copy(srci_h.at[pl.ds(base, _CH)], sidxs[b])
            pltpu.sync_copy(dsti_h.at[pl.ds(base, _CH)], didxs[b])
            pltpu.async_copy(vals_h.at[sidxs[b]], rows[b], sems[b])

        load_issue(0, 0)
        load_issue(1, 1)

        def body(j2, carry):
            for b in range(2):
                j = j2 * 2 + b
                pltpu.make_async_copy(vals_h.at[sidxs[b]], rows[b],
                                      sems[b]).wait()
                pltpu.sync_copy(rows[b], acc.at[didxs[b]], add=True)

                @pl.when(j + 2 < cpt)
                def _next():
                    load_issue(j + 2, b)
            return carry

        lax.fori_loop(0, cpt // 2, body, 0)
        plsc.subcore_barrier()
        pltpu.sync_copy(acc.at[pl.ds(sid * rpt, rpt)],
                        out_h.at[cid, pl.ds(sid * rpt, rpt)])

    return k(vals, srci, dsti, zeros)


def _deg_sc(dsti, ones_rows, zeros, etot):
    npad = _NPAD
    cpt = etot // (32 * _CH)
    ept = etot // 32
    rpt = npad // 16
    mesh = plsc.VectorSubcoreMesh(core_axis_name="c", subcore_axis_name="s")

    @functools.partial(
        pl.kernel,
        out_type=jax.ShapeDtypeStruct((2, npad, 16), jnp.float32),
        mesh=mesh,
        scratch_types=[
            pltpu.VMEM((_CH,), jnp.int32),
            pltpu.VMEM((_CH, 16), jnp.float32),
            pltpu.VMEM_SHARED((npad, 16), jnp.float32),
        ],
        compiler_params=pltpu.CompilerParams(use_tc_tiling_on_sc=False),
    )
    def k(dsti_h, ones_h, zeros_h, out_h, didx, rows, acc):
        cid = lax.axis_index("c")
        sid = lax.axis_index("s")
        wid = sid * 2 + cid
        pltpu.sync_copy(ones_h, rows)
        pltpu.sync_copy(zeros_h.at[pl.ds(sid * rpt, rpt)],
                        acc.at[pl.ds(sid * rpt, rpt)])
        plsc.subcore_barrier()

        def body(j, carry):
            base = wid * ept + j * _CH
            pltpu.sync_copy(dsti_h.at[pl.ds(base, _CH)], didx)
            pltpu.sync_copy(rows, acc.at[didx], add=True)
            return carry

        lax.fori_loop(0, cpt, body, 0)
        plsc.subcore_barrier()
        pltpu.sync_copy(acc.at[pl.ds(sid * rpt, rpt)],
                        out_h.at[cid, pl.ds(sid * rpt, rpt)])

    return k(dsti, ones_rows, zeros)


def _gat_edge_sc(asT, adT, cT, srci, dsti, zeros16, etot):
    npad = _NPAD
    cpt = etot // (32 * _CH)
    ept = etot // 32
    rpt = npad // 16
    mesh = plsc.VectorSubcoreMesh(core_axis_name="c", subcore_axis_name="s")

    @functools.partial(
        pl.kernel,
        out_type=[
            jax.ShapeDtypeStruct((etot, 16), jnp.float32),
            jax.ShapeDtypeStruct((2, npad, 16), jnp.float32),
        ],
        mesh=mesh,
        scratch_types=[
            [pltpu.VMEM((_CH,), jnp.int32)] * 2,
            [pltpu.VMEM((_CH,), jnp.int32)] * 2,
            [pltpu.VMEM((_CH, 16), jnp.float32)] * 2,
            [pltpu.VMEM((_CH, 16), jnp.float32)] * 2,
            [pltpu.VMEM((_CH, 16), jnp.float32)] * 2,
            [pltpu.VMEM((_CH, 16), jnp.float32)] * 2,
            pltpu.VMEM_SHARED((npad, 16), jnp.float32),
            [pltpu.SemaphoreType.DMA] * 2,
        ],
        compiler_params=pltpu.CompilerParams(use_tc_tiling_on_sc=False),
    )
    def k(as_h, ad_h, c_h, srci_h, dsti_h, zeros_h, ex_h, dn_h,
          sidxs, didxs, ars, brs, crs, exbs, acc, gsems):
        cid = lax.axis_index("c")
        sid = lax.axis_index("s")
        wid = sid * 2 + cid
        pltpu.sync_copy(zeros_h.at[pl.ds(sid * rpt, rpt)],
                        acc.at[pl.ds(sid * rpt, rpt)])
        plsc.subcore_barrier()
        mask = lax.iota(jnp.int32, 16) < 4

        def load_issue(j, b):
            base = wid * ept + j * _CH
            pltpu.sync_copy(srci_h.at[pl.ds(base, _CH)], sidxs[b])
            pltpu.sync_copy(dsti_h.at[pl.ds(base, _CH)], didxs[b])
            pltpu.async_copy(as_h.at[sidxs[b]], ars[b], gsems[b])
            pltpu.async_copy(ad_h.at[didxs[b]], brs[b], gsems[b])
            pltpu.async_copy(c_h.at[didxs[b]], crs[b], gsems[b])

        def wait_gathers(b):
            pltpu.make_async_copy(as_h.at[sidxs[b]], ars[b],
                                  gsems[b]).wait()
            pltpu.make_async_copy(ad_h.at[didxs[b]], brs[b],
                                  gsems[b]).wait()
            pltpu.make_async_copy(c_h.at[didxs[b]], crs[b],
                                  gsems[b]).wait()

        load_issue(0, 0)
        load_issue(1, 1)

        def body(j2, carry):
            for b in range(2):
                j = j2 * 2 + b
                base = wid * ept + j * _CH
                wait_gathers(b)
                ar, br, cr, exb = ars[b], brs[b], crs[b], exbs[b]

                @plsc.parallel_loop(0, _CH, 1, unroll=4)
                def inner(e):
                    pre = ar[e, :] + br[e, :]
                    alpha = jnp.maximum(pre, 0.2 * pre)
                    exv = jnp.exp(alpha - cr[e, :])
                    exb[e, :] = jnp.where(mask, exv, 0.0)

                pltpu.sync_copy(exb, ex_h.at[pl.ds(base, _CH)])
                pltpu.sync_copy(exb, acc.at[didxs[b]], add=True)

                @pl.when(j + 2 < cpt)
                def _next():
                    load_issue(j + 2, b)
            return carry

        lax.fori_loop(0, cpt // 2, body, 0)
        plsc.subcore_barrier()
        pltpu.sync_copy(acc.at[pl.ds(sid * rpt, rpt)],
                        dn_h.at[cid, pl.ds(sid * rpt, rpt)])

    return k(asT, adT, cT, srci, dsti, zeros16)


def _gat_agg_sc(xq0, xq1, xq2, xq3, ex, srci, dsti, zeros128, etot):
    npad = _NPAD
    ept = etot // 16
    cpt = ept // _CH
    rpt = npad // 16
    mesh = plsc.VectorSubcoreMesh(core_axis_name="c", subcore_axis_name="s")

    @functools.partial(
        pl.kernel,
        out_type=jax.ShapeDtypeStruct((2, 2, npad, 128), jnp.float32),
        mesh=mesh,
        scratch_types=[
            pltpu.VMEM((_CH,), jnp.int32),
            pltpu.VMEM((_CH,), jnp.int32),
            pltpu.VMEM((_CH,), jnp.int32),
            pltpu.VMEM((_CH,), jnp.int32),
            pltpu.VMEM((_CH, 32), jnp.float32),
            pltpu.VMEM((_CH, 32), jnp.float32),
            pltpu.VMEM((_CH, 16), jnp.float32),
            pltpu.VMEM((_CH, 128), jnp.float32),
            pltpu.VMEM((_CH, 128), jnp.float32),
            pltpu.VMEM_SHARED((npad, 128), jnp.float32),
            pltpu.SemaphoreType.DMA,
            pltpu.SemaphoreType.DMA,
            pltpu.SemaphoreType.DMA,
            pltpu.SemaphoreType.DMA,
        ],
        compiler_params=pltpu.CompilerParams(use_tc_tiling_on_sc=False),
    )
    def k(x0_h, x1_h, x2_h, x3_h, ex_h, srci_h, dsti_h, zeros_h, out_h,
          sidx0, sidx1, didx0, didx1, xg0, xg1, exg, scb0, scb1, acc,
          sem0, sem1, ssem0, ssem1):
        cid = lax.axis_index("c")
        sid = lax.axis_index("s")
        sidxs = (sidx0, sidx1)
        didxs = (didx0, didx1)
        xgs = (xg0, xg1)
        scbs = (scb0, scb1)
        sems = (sem0, sem1)
        ssems = (ssem0, ssem1)

        for p in range(2):
            xq_h = (x0_h, x2_h) if p == 0 else (x1_h, x3_h)
            pltpu.sync_copy(zeros_h.at[pl.ds(sid * rpt, rpt)],
                            acc.at[pl.ds(sid * rpt, rpt)])
            plsc.subcore_barrier()

            def load_sidx(j, b):
                base = sid * ept + j * _CH
                pltpu.sync_copy(srci_h.at[pl.ds(base, _CH)], sidxs[b])

            def load_didx(j, b):
                base = sid * ept + j * _CH
                pltpu.sync_copy(dsti_h.at[pl.ds(base, _CH)], didxs[b])

            def issue(b, q0_h, q1_h):
                @pl.when(cid == 0)
                def _a():
                    pltpu.async_copy(q0_h.at[sidxs[b]], xgs[b], sems[b])

                @pl.when(cid == 1)
                def _b():
                    pltpu.async_copy(q1_h.at[sidxs[b]], xgs[b], sems[b])

            def wait(b, q0_h, q1_h):
                @pl.when(cid == 0)
                def _a():
                    pltpu.make_async_copy(q0_h.at[sidxs[b]], xgs[b],
                                          sems[b]).wait()

                @pl.when(cid == 1)
                def _b():
                    pltpu.make_async_copy(q1_h.at[sidxs[b]], xgs[b],
                                          sems[b]).wait()

            q0_h, q1_h = xq_h
            load_sidx(0, 0)
            issue(0, q0_h, q1_h)
            load_sidx(1, 1)
            issue(1, q0_h, q1_h)

            def body(j2, carry):
                for b in range(2):
                    j = j2 * 2 + b
                    base = sid * ept + j * _CH
                    wait(b, q0_h, q1_h)
                    pltpu.sync_copy(ex_h.at[pl.ds(base, _CH)], exg)
                    xg = xgs[b]
                    scb = scbs[b]

                    @pl.when(j >= 2)
                    def _drain():
                        pltpu.make_async_copy(scb, acc.at[didxs[b]],
                                              ssems[b]).wait()

                    @plsc.parallel_loop(0, _CH, 1, unroll=4)
                    def inner(e):
                        exv = exg[e, :]
                        v0 = xg[e, pl.ds(0, 16)]
                        v1 = xg[e, pl.ds(16, 16)]
                        for hh in range(4):
                            w = exv[hh]
                            scb[e, pl.ds(hh * 32, 16)] = v0 * w
                            scb[e, pl.ds(hh * 32 + 16, 16)] = v1 * w

                    load_didx(j, b)
                    pltpu.async_copy(scb, acc.at[didxs[b]], ssems[b],
                                     add=True)

                    @pl.when(j + 2 < cpt)
                    def _next():
                        load_sidx(j + 2, b)
                        issue(b, q0_h, q1_h)
                return carry

            lax.fori_loop(0, cpt // 2, body, 0)
            for b in range(2):
                pltpu.make_async_copy(scbs[b], acc.at[didxs[b]],
                                      ssems[b]).wait()
            plsc.subcore_barrier()
            pltpu.sync_copy(acc.at[pl.ds(sid * rpt, rpt)],
                            out_h.at[cid, p, pl.ds(sid * rpt, rpt)])
            plsc.subcore_barrier()

    return k(xq0, xq1, xq2, xq3, ex, srci, dsti, zeros128)


def _gcn_dense_body(p_ref, dinv_ref, w_ref, b_ref, out_ref):
    s = (p_ref[0] + p_ref[1]) * dinv_ref[...]
    h = jnp.dot(s, w_ref[...], preferred_element_type=jnp.float32)
    h = jnp.maximum(h + b_ref[...], 0.0)
    out_ref[...] = h * dinv_ref[...]


def _gcn_dense(p, dinv, w, b, blk=1280):
    npad, fin = p.shape[1], p.shape[2]
    fout = w.shape[1]
    return pl.pallas_call(
        _gcn_dense_body,
        grid=(npad // blk,),
        in_specs=[
            pl.BlockSpec((2, blk, fin), lambda i: (0, i, 0)),
            pl.BlockSpec((blk, 1), lambda i: (i, 0)),
            pl.BlockSpec((fin, fout), lambda i: (0, 0)),
            pl.BlockSpec((1, fout), lambda i: (0, 0)),
        ],
        out_specs=pl.BlockSpec((blk, fout), lambda i: (i, 0)),
        out_shape=jax.ShapeDtypeStruct((npad, fout), jnp.float32),
    )(p, dinv, w, b)


def _gat_prep_body(p_ref, dinv_ref, w_ref, b_ref, avs_ref, avd_ref,
                   x2_ref, as_ref, ad_ref, c_ref):
    s = (p_ref[0] + p_ref[1]) * dinv_ref[...]
    x2 = jnp.dot(s, w_ref[...], preferred_element_type=jnp.float32)
    x2 = jnp.maximum(x2 + b_ref[...], 0.0)
    x2_ref[...] = x2
    a_s = jnp.dot(x2, avs_ref[...], preferred_element_type=jnp.float32)
    a_d = jnp.dot(x2, avd_ref[...], preferred_element_type=jnp.float32)
    pre = a_s + a_d
    c = jnp.maximum(pre, 0.2 * pre)
    z = jnp.zeros((a_s.shape[0], 12), jnp.float32)
    as_ref[...] = jnp.concatenate([a_s, z], axis=1)
    ad_ref[...] = jnp.concatenate([a_d, z], axis=1)
    c_ref[...] = jnp.concatenate([c, z], axis=1)


def _gat_prep(p, dinv, w, b, avs, avd, blk=1280):
    npad, fin = p.shape[1], p.shape[2]
    fout = w.shape[1]
    return pl.pallas_call(
        _gat_prep_body,
        grid=(npad // blk,),
        in_specs=[
            pl.BlockSpec((2, blk, fin), lambda i: (0, i, 0)),
            pl.BlockSpec((blk, 1), lambda i: (i, 0)),
            pl.BlockSpec((fin, fout), lambda i: (0, 0)),
            pl.BlockSpec((1, fout), lambda i: (0, 0)),
            pl.BlockSpec((fout, 4), lambda i: (0, 0)),
            pl.BlockSpec((fout, 4), lambda i: (0, 0)),
        ],
        out_specs=[
            pl.BlockSpec((blk, fout), lambda i: (i, 0)),
            pl.BlockSpec((blk, 16), lambda i: (i, 0)),
            pl.BlockSpec((blk, 16), lambda i: (i, 0)),
            pl.BlockSpec((blk, 16), lambda i: (i, 0)),
        ],
        out_shape=[
            jax.ShapeDtypeStruct((npad, fout), jnp.float32),
            jax.ShapeDtypeStruct((npad, 16), jnp.float32),
            jax.ShapeDtypeStruct((npad, 16), jnp.float32),
            jax.ShapeDtypeStruct((npad, 16), jnp.float32),
        ],
    )(p, dinv, w, b, avs, avd)


def _tail_body(a3_ref, dp_ref, w3s_ref, b3_ref, gw1_ref, gb1_ref, gw2_ref,
               gb2_ref, mw_ref, mb_ref, h_ref, g_ref, m_s, s_s, g_s):
    i = pl.program_id(0)
    nblk = pl.num_programs(0)

    @pl.when(i == 0)
    def _init():
        m_s[0, 0] = _NEG_INF
        s_s[0, 0] = 0.0
        g_s[...] = jnp.zeros_like(g_s)

    d = dp_ref[0] + dp_ref[1]
    r = 1.0 / (d + 1e-16)
    hid = h_ref.shape[1]
    acc = jnp.zeros((a3_ref.shape[1], hid), jnp.float32)
    for hh in range(4):
        sc = a3_ref[hh] * r[:, hh:hh + 1]
        acc = acc + jnp.dot(sc, w3s_ref[hh * 128:(hh + 1) * 128, :],
                            preferred_element_type=jnp.float32)
    h = acc * 0.25 + b3_ref[...]
    h_ref[...] = h

    z1 = jnp.maximum(
        jnp.dot(h, gw1_ref[...], preferred_element_type=jnp.float32)
        + gb1_ref[...], 0.0)
    z = jnp.dot(z1, gw2_ref[...], preferred_element_type=jnp.float32)
    z = z + gb2_ref[0, 0]

    blk_max = jnp.max(z)
    m_old = m_s[0, 0]
    m_new = jnp.maximum(m_old, blk_max)
    corr = jnp.exp(m_old - m_new)
    p = jnp.exp(z - m_new)
    s_s[0, 0] = s_s[0, 0] * corr + jnp.sum(p)
    g_s[...] = g_s[...] * corr + jnp.sum(p * h, axis=0, keepdims=True)
    m_s[0, 0] = m_new

    @pl.when(i == nblk - 1)
    def _fin():
        g = g_s[...] / s_s[0, 0]
        g = jnp.dot(g, mw_ref[...], preferred_element_type=jnp.float32)
        g_ref[...] = jnp.maximum(g + mb_ref[...], 0.0)


def _tail(a3, dp, w3s, b3, gw1, gb1, gw2, gb2, mw, mb, n, blk=400):
    hid = w3s.shape[1]
    return pl.pallas_call(
        _tail_body,
        grid=(n // blk,),
        in_specs=[
            pl.BlockSpec((4, blk, 128), lambda i: (0, i, 0)),
            pl.BlockSpec((2, blk, 16), lambda i: (0, i, 0)),
            pl.BlockSpec((512, hid), lambda i: (0, 0)),
            pl.BlockSpec((1, hid), lambda i: (0, 0)),
            pl.BlockSpec((hid, hid), lambda i: (0, 0)),
            pl.BlockSpec((1, hid), lambda i: (0, 0)),
            pl.BlockSpec((hid, 1), lambda i: (0, 0)),
            pl.BlockSpec((1, 1), lambda i: (0, 0)),
            pl.BlockSpec((hid, hid), lambda i: (0, 0)),
            pl.BlockSpec((1, hid), lambda i: (0, 0)),
        ],
        out_specs=[
            pl.BlockSpec((blk, hid), lambda i: (i, 0)),
            pl.BlockSpec((1, hid), lambda i: (0, 0)),
        ],
        out_shape=[
            jax.ShapeDtypeStruct((n, hid), jnp.float32),
            jax.ShapeDtypeStruct((1, hid), jnp.float32),
        ],
        scratch_shapes=[
            pltpu.SMEM((1, 1), jnp.float32),
            pltpu.SMEM((1, 1), jnp.float32),
            pltpu.VMEM((1, hid), jnp.float32),
        ],
    )(a3, dp, w3s, b3, gw1, gb1, gw2, gb2, mw, mb)


def kernel(x, edge_index, W1, b1, W2, b2, W3, att_src, att_dst, b3,
           gate_W1, gate_b1, gate_W2, gate_b2, mlp_W, mlp_b):
    n = x.shape[0]
    heads, hidden = att_src.shape
    in2 = W3.shape[0]
    npad = _NPAD

    loop = jnp.arange(n, dtype=jnp.int32)
    src = jnp.concatenate([edge_index[0], loop])
    dst = jnp.concatenate([edge_index[1], loop])
    e_real = src.shape[0]
    etot = ((e_real + 4095) // 4096) * 4096
    padn = etot - e_real
    srcp = jnp.concatenate([src, jnp.zeros((padn,), jnp.int32)])
    dstp = jnp.concatenate([dst, jnp.full((padn,), n, jnp.int32)])

    zeros16 = jnp.zeros((npad, 16), jnp.float32)
    zeros64 = jnp.zeros((npad, 64), jnp.float32)
    ones16 = jnp.ones((_CH, 16), jnp.float32)

    degp = _deg_sc(dstp, ones16, zeros16, etot)
    deg = degp[0, :, 0] + degp[1, :, 0]
    dinv = lax.rsqrt(jnp.maximum(deg, 1e-12))
    dinv2 = dinv[:, None]

    xs16 = jnp.zeros((npad, 16), jnp.float32)
    xs16 = xs16.at[:n, :3].set(x * dinv2[:n])
    agg1 = _segsum_sc(xs16, srcp, dstp, zeros16, 16, etot)
    w1p = jnp.zeros((16, 64), jnp.float32).at[:3].set(W1)
    x1s = _gcn_dense(agg1, dinv2, w1p, b1.reshape(1, 64))

    agg2 = _segsum_sc(x1s, srcp, dstp, zeros64, 64, etot)
    w3r = W3.reshape(in2, heads, hidden)
    avs = jnp.einsum("khj,hj->kh", w3r, att_src)
    avd = jnp.einsum("khj,hj->kh", w3r, att_dst)
    x2, asT, adT, cT = _gat_prep(agg2, dinv2, W2, b2.reshape(1, in2),
                                 avs, avd)

    exT, dpart = _gat_edge_sc(asT, adT, cT, srcp, dstp, zeros16, etot)
    zeros128 = jnp.zeros((npad, 128), jnp.float32)
    xq = [x2[:, 32 * q:32 * (q + 1)] for q in range(4)]
    aq = _gat_agg_sc(xq[0], xq[1], xq[2], xq[3], exT, srcp, dstp,
                     zeros128, etot)
    a3 = aq.reshape(4, npad, 4, 32).transpose(2, 1, 0, 3).reshape(
        4, npad, 128)

    w3s = w3r.transpose(1, 0, 2).reshape(heads * in2, hidden)
    h, g = _tail(a3, dpart, w3s, b3.reshape(1, hidden),
                 gate_W1, gate_b1.reshape(1, hidden),
                 gate_W2, gate_b2.reshape(1, 1),
                 mlp_W, mlp_b.reshape(1, hidden), n)
    return (g, h)


# speedup vs baseline: 26.0904x; 1.0137x over previous
import functools

import jax
import jax.numpy as jnp
from jax import lax
from jax.experimental import pallas as pl
from jax.experimental.pallas import tpu as pltpu
from jax.experimental.pallas import tpu_sc as plsc

_NEG_INF = float("-inf")
_NPAD = 10240
_CH = 128


def _segsum_sc(vals, srci, dsti, zeros, f, etot):
    npad = _NPAD
    cpt = etot // (32 * _CH)
    ept = etot // 32
    rpt = npad // 16
    mesh = plsc.VectorSubcoreMesh(core_axis_name="c", subcore_axis_name="s")

    @functools.partial(
        pl.kernel,
        out_type=jax.ShapeDtypeStruct((2, npad, f), jnp.float32),
        mesh=mesh,
        scratch_types=[
            [pltpu.VMEM((_CH,), jnp.int32)] * 3,
            [pltpu.VMEM((_CH,), jnp.int32)] * 3,
            [pltpu.VMEM((_CH, f), jnp.float32)] * 3,
            pltpu.VMEM_SHARED((npad, f), jnp.float32),
            [pltpu.SemaphoreType.DMA] * 3,
            [pltpu.SemaphoreType.DMA] * 3,
        ],
        compiler_params=pltpu.CompilerParams(use_tc_tiling_on_sc=False),
    )
    def k(vals_h, srci_h, dsti_h, zeros_h, out_h,
          sidxs, didxs, rows, acc, gsems, ssems):
        cid = lax.axis_index("c")
        sid = lax.axis_index("s")
        wid = sid * 2 + cid
        pltpu.sync_copy(zeros_h.at[pl.ds(sid * rpt, rpt)],
                        acc.at[pl.ds(sid * rpt, rpt)])
        plsc.subcore_barrier()

        def load_issue(j, b):
            base = wid * ept + j * _CH
            pltpu.sync_copy(srci_h.at[pl.ds(base, _CH)], sidxs[b])
            pltpu.sync_copy(dsti_h.at[pl.ds(base, _CH)], didxs[b])
            pltpu.async_copy(vals_h.at[sidxs[b]], rows[b], gsems[b])

        def wait_scat(b):
            pltpu.make_async_copy(rows[b], acc.at[didxs[b]],
                                  ssems[b]).wait()

        load_issue(0, 0)
        load_issue(1, 1)

        def body(j3, carry):
            for b3 in range(3):
                j = j3 * 3 + b3
                b = b3 % 3
                bn = (b3 + 2) % 3
                pltpu.make_async_copy(vals_h.at[sidxs[b]], rows[b],
                                      gsems[b]).wait()
                pltpu.async_copy(rows[b], acc.at[didxs[b]], ssems[b],
                                 add=True)

                @pl.when(j + 2 < cpt)
                def _next():
                    @pl.when(j >= 1)
                    def _w():
                        wait_scat(bn)

                    load_issue(j + 2, bn)
            return carry

        lax.fori_loop(0, cpt // 3, body, 0)
        for b in range(3):
            wait_scat(b)
        plsc.subcore_barrier()
        pltpu.sync_copy(acc.at[pl.ds(sid * rpt, rpt)],
                        out_h.at[cid, pl.ds(sid * rpt, rpt)])

    return k(vals, srci, dsti, zeros)


def _deg_sc(dsti, ones_rows, zeros, etot):
    npad = _NPAD
    cpt = etot // (32 * _CH)
    ept = etot // 32
    rpt = npad // 16
    mesh = plsc.VectorSubcoreMesh(core_axis_name="c", subcore_axis_name="s")

    @functools.partial(
        pl.kernel,
        out_type=jax.ShapeDtypeStruct((2, npad, 16), jnp.float32),
        mesh=mesh,
        scratch_types=[
            pltpu.VMEM((_CH,), jnp.int32),
            pltpu.VMEM((_CH, 16), jnp.float32),
            pltpu.VMEM_SHARED((npad, 16), jnp.float32),
        ],
        compiler_params=pltpu.CompilerParams(use_tc_tiling_on_sc=False),
    )
    def k(dsti_h, ones_h, zeros_h, out_h, didx, rows, acc):
        cid = lax.axis_index("c")
        sid = lax.axis_index("s")
        wid = sid * 2 + cid
        pltpu.sync_copy(ones_h, rows)
        pltpu.sync_copy(zeros_h.at[pl.ds(sid * rpt, rpt)],
                        acc.at[pl.ds(sid * rpt, rpt)])
        plsc.subcore_barrier()

        def body(j, carry):
            base = wid * ept + j * _CH
            pltpu.sync_copy(dsti_h.at[pl.ds(base, _CH)], didx)
            pltpu.sync_copy(rows, acc.at[didx], add=True)
            return carry

        lax.fori_loop(0, cpt, body, 0)
        plsc.subcore_barrier()
        pltpu.sync_copy(acc.at[pl.ds(sid * rpt, rpt)],
                        out_h.at[cid, pl.ds(sid * rpt, rpt)])

    return k(dsti, ones_rows, zeros)


def _gat_edge_sc(asT, adT, cT, srci, dsti, zeros16, etot):
    npad = _NPAD
    cpt = etot // (32 * _CH)
    ept = etot // 32
    rpt = npad // 16
    mesh = plsc.VectorSubcoreMesh(core_axis_name="c", subcore_axis_name="s")

    @functools.partial(
        pl.kernel,
        out_type=[
            jax.ShapeDtypeStruct((etot, 16), jnp.float32),
            jax.ShapeDtypeStruct((2, npad, 16), jnp.float32),
        ],
        mesh=mesh,
        scratch_types=[
            [pltpu.VMEM((_CH,), jnp.int32)] * 2,
            [pltpu.VMEM((_CH,), jnp.int32)] * 2,
            [pltpu.VMEM((_CH, 16), jnp.float32)] * 2,
            [pltpu.VMEM((_CH, 16), jnp.float32)] * 2,
            [pltpu.VMEM((_CH, 16), jnp.float32)] * 2,
            [pltpu.VMEM((_CH, 16), jnp.float32)] * 2,
            pltpu.VMEM_SHARED((npad, 16), jnp.float32),
            [pltpu.SemaphoreType.DMA] * 2,
        ],
        compiler_params=pltpu.CompilerParams(use_tc_tiling_on_sc=False),
    )
    def k(as_h, ad_h, c_h, srci_h, dsti_h, zeros_h, ex_h, dn_h,
          sidxs, didxs, ars, brs, crs, exbs, acc, gsems):
        cid = lax.axis_index("c")
        sid = lax.axis_index("s")
        wid = sid * 2 + cid
        pltpu.sync_copy(zeros_h.at[pl.ds(sid * rpt, rpt)],
                        acc.at[pl.ds(sid * rpt, rpt)])
        plsc.subcore_barrier()
        mask = lax.iota(jnp.int32, 16) < 4

        def load_issue(j, b):
            base = wid * ept + j * _CH
            pltpu.sync_copy(srci_h.at[pl.ds(base, _CH)], sidxs[b])
            pltpu.sync_copy(dsti_h.at[pl.ds(base, _CH)], didxs[b])
            pltpu.async_copy(as_h.at[sidxs[b]], ars[b], gsems[b])
            pltpu.async_copy(ad_h.at[didxs[b]], brs[b], gsems[b])
            pltpu.async_copy(c_h.at[didxs[b]], crs[b], gsems[b])

        def wait_gathers(b):
            pltpu.make_async_copy(as_h.at[sidxs[b]], ars[b],
                                  gsems[b]).wait()
            pltpu.make_async_copy(ad_h.at[didxs[b]], brs[b],
                                  gsems[b]).wait()
            pltpu.make_async_copy(c_h.at[didxs[b]], crs[b],
                                  gsems[b]).wait()

        load_issue(0, 0)
        load_issue(1, 1)

        def body(j2, carry):
            for b in range(2):
                j = j2 * 2 + b
                base = wid * ept + j * _CH
                wait_gathers(b)
                ar, br, cr, exb = ars[b], brs[b], crs[b], exbs[b]

                @plsc.parallel_loop(0, _CH, 1, unroll=4)
                def inner(e):
                    pre = ar[e, :] + br[e, :]
                    alpha = jnp.maximum(pre, 0.2 * pre)
                    exv = jnp.exp(alpha - cr[e, :])
                    exb[e, :] = jnp.where(mask, exv, 0.0)

                pltpu.sync_copy(exb, ex_h.at[pl.ds(base, _CH)])
                pltpu.sync_copy(exb, acc.at[didxs[b]], add=True)

                @pl.when(j + 2 < cpt)
                def _next():
                    load_issue(j + 2, b)
            return carry

        lax.fori_loop(0, cpt // 2, body, 0)
        plsc.subcore_barrier()
        pltpu.sync_copy(acc.at[pl.ds(sid * rpt, rpt)],
                        dn_h.at[cid, pl.ds(sid * rpt, rpt)])

    return k(asT, adT, cT, srci, dsti, zeros16)


def _gat_agg_sc(xq0, xq1, xq2, xq3, ex, srci, dsti, zeros128, etot):
    npad = _NPAD
    ept = etot // 16
    cpt = ept // _CH
    rpt = npad // 16
    mesh = plsc.VectorSubcoreMesh(core_axis_name="c", subcore_axis_name="s")

    @functools.partial(
        pl.kernel,
        out_type=jax.ShapeDtypeStruct((2, 2, npad, 128), jnp.float32),
        mesh=mesh,
        scratch_types=[
            pltpu.VMEM((_CH,), jnp.int32),
            pltpu.VMEM((_CH,), jnp.int32),
            pltpu.VMEM((_CH,), jnp.int32),
            pltpu.VMEM((_CH,), jnp.int32),
            pltpu.VMEM((_CH, 32), jnp.float32),
            pltpu.VMEM((_CH, 32), jnp.float32),
            pltpu.VMEM((_CH, 16), jnp.float32),
            pltpu.VMEM((_CH, 128), jnp.float32),
            pltpu.VMEM((_CH, 128), jnp.float32),
            pltpu.VMEM_SHARED((npad, 128), jnp.float32),
            pltpu.SemaphoreType.DMA,
            pltpu.SemaphoreType.DMA,
            pltpu.SemaphoreType.DMA,
            pltpu.SemaphoreType.DMA,
        ],
        compiler_params=pltpu.CompilerParams(use_tc_tiling_on_sc=False),
    )
    def k(x0_h, x1_h, x2_h, x3_h, ex_h, srci_h, dsti_h, zeros_h, out_h,
          sidx0, sidx1, didx0, didx1, xg0, xg1, exg, scb0, scb1, acc,
          sem0, sem1, ssem0, ssem1):
        cid = lax.axis_index("c")
        sid = lax.axis_index("s")
        sidxs = (sidx0, sidx1)
        didxs = (didx0, didx1)
        xgs = (xg0, xg1)
        scbs = (scb0, scb1)
        sems = (sem0, sem1)
        ssems = (ssem0, ssem1)

        for p in range(2):
            xq_h = (x0_h, x2_h) if p == 0 else (x1_h, x3_h)
            pltpu.sync_copy(zeros_h.at[pl.ds(sid * rpt, rpt)],
                            acc.at[pl.ds(sid * rpt, rpt)])
            plsc.subcore_barrier()

            def load_sidx(j, b):
                base = sid * ept + j * _CH
                pltpu.sync_copy(srci_h.at[pl.ds(base, _CH)], sidxs[b])

            def load_didx(j, b):
                base = sid * ept + j * _CH
                pltpu.sync_copy(dsti_h.at[pl.ds(base, _CH)], didxs[b])

            def issue(b, q0_h, q1_h):
                @pl.when(cid == 0)
                def _a():
                    pltpu.async_copy(q0_h.at[sidxs[b]], xgs[b], sems[b])

                @pl.when(cid == 1)
                def _b():
                    pltpu.async_copy(q1_h.at[sidxs[b]], xgs[b], sems[b])

            def wait(b, q0_h, q1_h):
                @pl.when(cid == 0)
                def _a():
                    pltpu.make_async_copy(q0_h.at[sidxs[b]], xgs[b],
                                          sems[b]).wait()

                @pl.when(cid == 1)
                def _b():
                    pltpu.make_async_copy(q1_h.at[sidxs[b]], xgs[b],
                                          sems[b]).wait()

            q0_h, q1_h = xq_h
            load_sidx(0, 0)
            issue(0, q0_h, q1_h)
            load_sidx(1, 1)
            issue(1, q0_h, q1_h)

            def body(j2, carry):
                for b in range(2):
                    j = j2 * 2 + b
                    base = sid * ept + j * _CH
                    wait(b, q0_h, q1_h)
                    pltpu.sync_copy(ex_h.at[pl.ds(base, _CH)], exg)
                    xg = xgs[b]
                    scb = scbs[b]

                    @pl.when(j >= 2)
                    def _drain():
                        pltpu.make_async_copy(scb, acc.at[didxs[b]],
                                              ssems[b]).wait()

                    @plsc.parallel_loop(0, _CH, 1, unroll=8)
                    def inner(e):
                        exv = exg[e, :]
                        v0 = xg[e, pl.ds(0, 16)]
                        v1 = xg[e, pl.ds(16, 16)]
                        for hh in range(4):
                            w = exv[hh]
                            scb[e, pl.ds(hh * 32, 16)] = v0 * w
                            scb[e, pl.ds(hh * 32 + 16, 16)] = v1 * w

                    load_didx(j, b)
                    pltpu.async_copy(scb, acc.at[didxs[b]], ssems[b],
                                     add=True)

                    @pl.when(j + 2 < cpt)
                    def _next():
                        load_sidx(j + 2, b)
                        issue(b, q0_h, q1_h)
                return carry

            lax.fori_loop(0, cpt // 2, body, 0)
            for b in range(2):
                pltpu.make_async_copy(scbs[b], acc.at[didxs[b]],
                                      ssems[b]).wait()
            plsc.subcore_barrier()
            pltpu.sync_copy(acc.at[pl.ds(sid * rpt, rpt)],
                            out_h.at[cid, p, pl.ds(sid * rpt, rpt)])
            plsc.subcore_barrier()

    return k(xq0, xq1, xq2, xq3, ex, srci, dsti, zeros128)


def _gcn_dense_body(p_ref, dinv_ref, w_ref, b_ref, out_ref):
    s = (p_ref[0] + p_ref[1]) * dinv_ref[...]
    h = jnp.dot(s, w_ref[...], preferred_element_type=jnp.float32)
    h = jnp.maximum(h + b_ref[...], 0.0)
    out_ref[...] = h * dinv_ref[...]


def _gcn_dense(p, dinv, w, b, blk=1280):
    npad, fin = p.shape[1], p.shape[2]
    fout = w.shape[1]
    return pl.pallas_call(
        _gcn_dense_body,
        grid=(npad // blk,),
        in_specs=[
            pl.BlockSpec((2, blk, fin), lambda i: (0, i, 0)),
            pl.BlockSpec((blk, 1), lambda i: (i, 0)),
            pl.BlockSpec((fin, fout), lambda i: (0, 0)),
            pl.BlockSpec((1, fout), lambda i: (0, 0)),
        ],
        out_specs=pl.BlockSpec((blk, fout), lambda i: (i, 0)),
        out_shape=jax.ShapeDtypeStruct((npad, fout), jnp.float32),
    )(p, dinv, w, b)


def _gat_prep_body(p_ref, dinv_ref, w_ref, b_ref, avs_ref, avd_ref,
                   x2_ref, as_ref, ad_ref, c_ref):
    s = (p_ref[0] + p_ref[1]) * dinv_ref[...]
    x2 = jnp.dot(s, w_ref[...], preferred_element_type=jnp.float32)
    x2 = jnp.maximum(x2 + b_ref[...], 0.0)
    x2_ref[...] = x2
    a_s = jnp.dot(x2, avs_ref[...], preferred_element_type=jnp.float32)
    a_d = jnp.dot(x2, avd_ref[...], preferred_element_type=jnp.float32)
    pre = a_s + a_d
    c = jnp.maximum(pre, 0.2 * pre)
    z = jnp.zeros((a_s.shape[0], 12), jnp.float32)
    as_ref[...] = jnp.concatenate([a_s, z], axis=1)
    ad_ref[...] = jnp.concatenate([a_d, z], axis=1)
    c_ref[...] = jnp.concatenate([c, z], axis=1)


def _gat_prep(p, dinv, w, b, avs, avd, blk=1280):
    npad, fin = p.shape[1], p.shape[2]
    fout = w.shape[1]
    return pl.pallas_call(
        _gat_prep_body,
        grid=(npad // blk,),
        in_specs=[
            pl.BlockSpec((2, blk, fin), lambda i: (0, i, 0)),
            pl.BlockSpec((blk, 1), lambda i: (i, 0)),
            pl.BlockSpec((fin, fout), lambda i: (0, 0)),
            pl.BlockSpec((1, fout), lambda i: (0, 0)),
            pl.BlockSpec((fout, 4), lambda i: (0, 0)),
            pl.BlockSpec((fout, 4), lambda i: (0, 0)),
        ],
        out_specs=[
            pl.BlockSpec((blk, fout), lambda i: (i, 0)),
            pl.BlockSpec((blk, 16), lambda i: (i, 0)),
            pl.BlockSpec((blk, 16), lambda i: (i, 0)),
            pl.BlockSpec((blk, 16), lambda i: (i, 0)),
        ],
        out_shape=[
            jax.ShapeDtypeStruct((npad, fout), jnp.float32),
            jax.ShapeDtypeStruct((npad, 16), jnp.float32),
            jax.ShapeDtypeStruct((npad, 16), jnp.float32),
            jax.ShapeDtypeStruct((npad, 16), jnp.float32),
        ],
    )(p, dinv, w, b, avs, avd)


def _tail_body(a3_ref, dp_ref, w3s_ref, b3_ref, gw1_ref, gb1_ref, gw2_ref,
               gb2_ref, mw_ref, mb_ref, h_ref, g_ref, m_s, s_s, g_s):
    i = pl.program_id(0)
    nblk = pl.num_programs(0)

    @pl.when(i == 0)
    def _init():
        m_s[0, 0] = _NEG_INF
        s_s[0, 0] = 0.0
        g_s[...] = jnp.zeros_like(g_s)

    d = dp_ref[0] + dp_ref[1]
    r = 1.0 / (d + 1e-16)
    hid = h_ref.shape[1]
    acc = jnp.zeros((a3_ref.shape[1], hid), jnp.float32)
    for hh in range(4):
        sc = a3_ref[hh] * r[:, hh:hh + 1]
        acc = acc + jnp.dot(sc, w3s_ref[hh * 128:(hh + 1) * 128, :],
                            preferred_element_type=jnp.float32)
    h = acc * 0.25 + b3_ref[...]
    h_ref[...] = h

    z1 = jnp.maximum(
        jnp.dot(h, gw1_ref[...], preferred_element_type=jnp.float32)
        + gb1_ref[...], 0.0)
    z = jnp.dot(z1, gw2_ref[...], preferred_element_type=jnp.float32)
    z = z + gb2_ref[0, 0]

    blk_max = jnp.max(z)
    m_old = m_s[0, 0]
    m_new = jnp.maximum(m_old, blk_max)
    corr = jnp.exp(m_old - m_new)
    p = jnp.exp(z - m_new)
    s_s[0, 0] = s_s[0, 0] * corr + jnp.sum(p)
    g_s[...] = g_s[...] * corr + jnp.sum(p * h, axis=0, keepdims=True)
    m_s[0, 0] = m_new

    @pl.when(i == nblk - 1)
    def _fin():
        g = g_s[...] / s_s[0, 0]
        g = jnp.dot(g, mw_ref[...], preferred_element_type=jnp.float32)
        g_ref[...] = jnp.maximum(g + mb_ref[...], 0.0)


def _tail(a3, dp, w3s, b3, gw1, gb1, gw2, gb2, mw, mb, n, blk=400):
    hid = w3s.shape[1]
    return pl.pallas_call(
        _tail_body,
        grid=(n // blk,),
        in_specs=[
            pl.BlockSpec((4, blk, 128), lambda i: (0, i, 0)),
            pl.BlockSpec((2, blk, 16), lambda i: (0, i, 0)),
            pl.BlockSpec((512, hid), lambda i: (0, 0)),
            pl.BlockSpec((1, hid), lambda i: (0, 0)),
            pl.BlockSpec((hid, hid), lambda i: (0, 0)),
            pl.BlockSpec((1, hid), lambda i: (0, 0)),
            pl.BlockSpec((hid, 1), lambda i: (0, 0)),
            pl.BlockSpec((1, 1), lambda i: (0, 0)),
            pl.BlockSpec((hid, hid), lambda i: (0, 0)),
            pl.BlockSpec((1, hid), lambda i: (0, 0)),
        ],
        out_specs=[
            pl.BlockSpec((blk, hid), lambda i: (i, 0)),
            pl.BlockSpec((1, hid), lambda i: (0, 0)),
        ],
        out_shape=[
            jax.ShapeDtypeStruct((n, hid), jnp.float32),
            jax.ShapeDtypeStruct((1, hid), jnp.float32),
        ],
        scratch_shapes=[
            pltpu.SMEM((1, 1), jnp.float32),
            pltpu.SMEM((1, 1), jnp.float32),
            pltpu.VMEM((1, hid), jnp.float32),
        ],
    )(a3, dp, w3s, b3, gw1, gb1, gw2, gb2, mw, mb)


def kernel(x, edge_index, W1, b1, W2, b2, W3, att_src, att_dst, b3,
           gate_W1, gate_b1, gate_W2, gate_b2, mlp_W, mlp_b):
    n = x.shape[0]
    heads, hidden = att_src.shape
    in2 = W3.shape[0]
    npad = _NPAD

    loop = jnp.arange(n, dtype=jnp.int32)
    src = jnp.concatenate([edge_index[0], loop])
    dst = jnp.concatenate([edge_index[1], loop])
    e_real = src.shape[0]
    etot = ((e_real + 4095) // 4096) * 4096
    padn = etot - e_real
    srcp = jnp.concatenate([src, jnp.zeros((padn,), jnp.int32)])
    dstp = jnp.concatenate([dst, jnp.full((padn,), n, jnp.int32)])

    zeros16 = jnp.zeros((npad, 16), jnp.float32)
    zeros64 = jnp.zeros((npad, 64), jnp.float32)
    ones16 = jnp.ones((_CH, 16), jnp.float32)

    degp = _deg_sc(dstp, ones16, zeros16, etot)
    deg = degp[0, :, 0] + degp[1, :, 0]
    dinv = lax.rsqrt(jnp.maximum(deg, 1e-12))
    dinv2 = dinv[:, None]

    xs16 = jnp.zeros((npad, 16), jnp.float32)
    xs16 = xs16.at[:n, :3].set(x * dinv2[:n])
    agg1 = _segsum_sc(xs16, srcp, dstp, zeros16, 16, etot)
    w1p = jnp.zeros((16, 64), jnp.float32).at[:3].set(W1)
    x1s = _gcn_dense(agg1, dinv2, w1p, b1.reshape(1, 64))

    agg2 = _segsum_sc(x1s, srcp, dstp, zeros64, 64, etot)
    w3r = W3.reshape(in2, heads, hidden)
    avs = jnp.einsum("khj,hj->kh", w3r, att_src)
    avd = jnp.einsum("khj,hj->kh", w3r, att_dst)
    x2, asT, adT, cT = _gat_prep(agg2, dinv2, W2, b2.reshape(1, in2),
                                 avs, avd)

    exT, dpart = _gat_edge_sc(asT, adT, cT, srcp, dstp, zeros16, etot)
    zeros128 = jnp.zeros((npad, 128), jnp.float32)
    xq = [x2[:, 32 * q:32 * (q + 1)] for q in range(4)]
    aq = _gat_agg_sc(xq[0], xq[1], xq[2], xq[3], exT, srcp, dstp,
                     zeros128, etot)
    a3 = aq.reshape(4, npad, 4, 32).transpose(2, 1, 0, 3).reshape(
        4, npad, 128)

    w3s = w3r.transpose(1, 0, 2).reshape(heads * in2, hidden)
    h, g = _tail(a3, dpart, w3s, b3.reshape(1, hidden),
                 gate_W1, gate_b1.reshape(1, hidden),
                 gate_W2, gate_b2.reshape(1, 1),
                 mlp_W, mlp_b.reshape(1, hidden), n)
    return (g, h)


# speedup vs baseline: 29.8688x; 1.1448x over previous
import functools

import jax
import jax.numpy as jnp
from jax import lax
from jax.experimental import pallas as pl
from jax.experimental.pallas import tpu as pltpu
from jax.experimental.pallas import tpu_sc as plsc

_NEG_INF = float("-inf")
_NPAD = 10240
_CH = 128


def _segsum_sc(vals, srci, dsti, zeros, f, etot):
    npad = _NPAD
    cpt = etot // (32 * _CH)
    ept = etot // 32
    rpt = npad // 16
    mesh = plsc.VectorSubcoreMesh(core_axis_name="c", subcore_axis_name="s")

    @functools.partial(
        pl.kernel,
        out_type=jax.ShapeDtypeStruct((2, npad, f), jnp.float32),
        mesh=mesh,
        scratch_types=[
            [pltpu.VMEM((_CH,), jnp.int32)] * 3,
            [pltpu.VMEM((_CH,), jnp.int32)] * 3,
            [pltpu.VMEM((_CH, f), jnp.float32)] * 3,
            pltpu.VMEM_SHARED((npad, f), jnp.float32),
            [pltpu.SemaphoreType.DMA] * 3,
            [pltpu.SemaphoreType.DMA] * 3,
        ],
        compiler_params=pltpu.CompilerParams(use_tc_tiling_on_sc=False),
    )
    def k(vals_h, srci_h, dsti_h, zeros_h, out_h,
          sidxs, didxs, rows, acc, gsems, ssems):
        cid = lax.axis_index("c")
        sid = lax.axis_index("s")
        wid = sid * 2 + cid
        pltpu.sync_copy(zeros_h.at[pl.ds(sid * rpt, rpt)],
                        acc.at[pl.ds(sid * rpt, rpt)])
        plsc.subcore_barrier()

        def load_issue(j, b):
            base = wid * ept + j * _CH
            pltpu.sync_copy(srci_h.at[pl.ds(base, _CH)], sidxs[b])
            pltpu.sync_copy(dsti_h.at[pl.ds(base, _CH)], didxs[b])
            pltpu.async_copy(vals_h.at[sidxs[b]], rows[b], gsems[b])

        def wait_scat(b):
            pltpu.make_async_copy(rows[b], acc.at[didxs[b]],
                                  ssems[b]).wait()

        load_issue(0, 0)
        load_issue(1, 1)

        def body(j3, carry):
            for b3 in range(3):
                j = j3 * 3 + b3
                b = b3 % 3
                bn = (b3 + 2) % 3
                pltpu.make_async_copy(vals_h.at[sidxs[b]], rows[b],
                                      gsems[b]).wait()
                pltpu.async_copy(rows[b], acc.at[didxs[b]], ssems[b],
                                 add=True)

                @pl.when(j + 2 < cpt)
                def _next():
                    @pl.when(j >= 1)
                    def _w():
                        wait_scat(bn)

                    load_issue(j + 2, bn)
            return carry

        lax.fori_loop(0, cpt // 3, body, 0)
        for b in range(3):
            wait_scat(b)
        plsc.subcore_barrier()
        pltpu.sync_copy(acc.at[pl.ds(sid * rpt, rpt)],
                        out_h.at[cid, pl.ds(sid * rpt, rpt)])

    return k(vals, srci, dsti, zeros)


def _deg_sc(dsti, ones_rows, zeros, etot):
    npad = _NPAD
    cpt = etot // (32 * _CH)
    ept = etot // 32
    rpt = npad // 16
    mesh = plsc.VectorSubcoreMesh(core_axis_name="c", subcore_axis_name="s")

    @functools.partial(
        pl.kernel,
        out_type=jax.ShapeDtypeStruct((2, npad, 16), jnp.float32),
        mesh=mesh,
        scratch_types=[
            pltpu.VMEM((_CH,), jnp.int32),
            pltpu.VMEM((_CH, 16), jnp.float32),
            pltpu.VMEM_SHARED((npad, 16), jnp.float32),
        ],
        compiler_params=pltpu.CompilerParams(use_tc_tiling_on_sc=False),
    )
    def k(dsti_h, ones_h, zeros_h, out_h, didx, rows, acc):
        cid = lax.axis_index("c")
        sid = lax.axis_index("s")
        wid = sid * 2 + cid
        pltpu.sync_copy(ones_h, rows)
        pltpu.sync_copy(zeros_h.at[pl.ds(sid * rpt, rpt)],
                        acc.at[pl.ds(sid * rpt, rpt)])
        plsc.subcore_barrier()

        def body(j, carry):
            base = wid * ept + j * _CH
            pltpu.sync_copy(dsti_h.at[pl.ds(base, _CH)], didx)
            pltpu.sync_copy(rows, acc.at[didx], add=True)
            return carry

        lax.fori_loop(0, cpt, body, 0)
        plsc.subcore_barrier()
        pltpu.sync_copy(acc.at[pl.ds(sid * rpt, rpt)],
                        out_h.at[cid, pl.ds(sid * rpt, rpt)])

    return k(dsti, ones_rows, zeros)


def _gat_edge_sc(asT, adT, cT, srci, dsti, zeros16, etot):
    npad = _NPAD
    cpt = etot // (32 * _CH)
    ept = etot // 32
    rpt = npad // 16
    mesh = plsc.VectorSubcoreMesh(core_axis_name="c", subcore_axis_name="s")

    @functools.partial(
        pl.kernel,
        out_type=[
            jax.ShapeDtypeStruct((etot, 16), jnp.float32),
            jax.ShapeDtypeStruct((2, npad, 16), jnp.float32),
        ],
        mesh=mesh,
        scratch_types=[
            [pltpu.VMEM((_CH,), jnp.int32)] * 2,
            [pltpu.VMEM((_CH,), jnp.int32)] * 2,
            [pltpu.VMEM((_CH, 16), jnp.float32)] * 2,
            [pltpu.VMEM((_CH, 16), jnp.float32)] * 2,
            [pltpu.VMEM((_CH, 16), jnp.float32)] * 2,
            [pltpu.VMEM((_CH, 16), jnp.float32)] * 2,
            pltpu.VMEM_SHARED((npad, 16), jnp.float32),
            [pltpu.SemaphoreType.DMA] * 2,
        ],
        compiler_params=pltpu.CompilerParams(use_tc_tiling_on_sc=False),
    )
    def k(as_h, ad_h, c_h, srci_h, dsti_h, zeros_h, ex_h, dn_h,
          sidxs, didxs, ars, brs, crs, exbs, acc, gsems):
        cid = lax.axis_index("c")
        sid = lax.axis_index("s")
        wid = sid * 2 + cid
        pltpu.sync_copy(zeros_h.at[pl.ds(sid * rpt, rpt)],
                        acc.at[pl.ds(sid * rpt, rpt)])
        plsc.subcore_barrier()
        mask = lax.iota(jnp.int32, 16) < 4

        def load_issue(j, b):
            base = wid * ept + j * _CH
            pltpu.sync_copy(srci_h.at[pl.ds(base, _CH)], sidxs[b])
            pltpu.sync_copy(dsti_h.at[pl.ds(base, _CH)], didxs[b])
            pltpu.async_copy(as_h.at[sidxs[b]], ars[b], gsems[b])
            pltpu.async_copy(ad_h.at[didxs[b]], brs[b], gsems[b])
            pltpu.async_copy(c_h.at[didxs[b]], crs[b], gsems[b])

        def wait_gathers(b):
            pltpu.make_async_copy(as_h.at[sidxs[b]], ars[b],
                                  gsems[b]).wait()
            pltpu.make_async_copy(ad_h.at[didxs[b]], brs[b],
                                  gsems[b]).wait()
            pltpu.make_async_copy(c_h.at[didxs[b]], crs[b],
                                  gsems[b]).wait()

        load_issue(0, 0)
        load_issue(1, 1)

        def body(j2, carry):
            for b in range(2):
                j = j2 * 2 + b
                base = wid * ept + j * _CH
                wait_gathers(b)
                ar, br, cr, exb = ars[b], brs[b], crs[b], exbs[b]

                @plsc.parallel_loop(0, _CH, 1, unroll=4)
                def inner(e):
                    pre = ar[e, :] + br[e, :]
                    alpha = jnp.maximum(pre, 0.2 * pre)
                    exv = jnp.exp(alpha - cr[e, :])
                    exb[e, :] = jnp.where(mask, exv, 0.0)

                pltpu.sync_copy(exb, ex_h.at[pl.ds(base, _CH)])
                pltpu.sync_copy(exb, acc.at[didxs[b]], add=True)

                @pl.when(j + 2 < cpt)
                def _next():
                    load_issue(j + 2, b)
            return carry

        lax.fori_loop(0, cpt // 2, body, 0)
        plsc.subcore_barrier()
        pltpu.sync_copy(acc.at[pl.ds(sid * rpt, rpt)],
                        dn_h.at[cid, pl.ds(sid * rpt, rpt)])

    return k(asT, adT, cT, srci, dsti, zeros16)


def _gat_agg_sc(xq0, xq1, xq2, xq3, ex, srci, dsti, zeros128, etot):
    npad = _NPAD
    ept = etot // 16
    cpt = ept // _CH
    rpt = npad // 16
    mesh = plsc.VectorSubcoreMesh(core_axis_name="c", subcore_axis_name="s")

    @functools.partial(
        pl.kernel,
        out_type=jax.ShapeDtypeStruct((2, 2, npad, 128), jnp.float32),
        mesh=mesh,
        scratch_types=[
            pltpu.VMEM((_CH,), jnp.int32),
            pltpu.VMEM((_CH,), jnp.int32),
            pltpu.VMEM((_CH,), jnp.int32),
            pltpu.VMEM((_CH,), jnp.int32),
            pltpu.VMEM((_CH, 32), jnp.float32),
            pltpu.VMEM((_CH, 32), jnp.float32),
            [pltpu.VMEM((_CH, 16), jnp.float32)] * 2,
            pltpu.VMEM((_CH, 128), jnp.float32),
            pltpu.VMEM((_CH, 128), jnp.float32),
            pltpu.VMEM_SHARED((npad, 128), jnp.float32),
            pltpu.SemaphoreType.DMA,
            pltpu.SemaphoreType.DMA,
            pltpu.SemaphoreType.DMA,
            pltpu.SemaphoreType.DMA,
            [pltpu.SemaphoreType.DMA] * 2,
        ],
        compiler_params=pltpu.CompilerParams(use_tc_tiling_on_sc=False),
    )
    def k(x0_h, x1_h, x2_h, x3_h, ex_h, srci_h, dsti_h, zeros_h, out_h,
          sidx0, sidx1, didx0, didx1, xg0, xg1, exgs, scb0, scb1, acc,
          sem0, sem1, ssem0, ssem1, esems):
        cid = lax.axis_index("c")
        sid = lax.axis_index("s")
        sidxs = (sidx0, sidx1)
        didxs = (didx0, didx1)
        xgs = (xg0, xg1)
        scbs = (scb0, scb1)
        sems = (sem0, sem1)
        ssems = (ssem0, ssem1)

        for p in range(2):
            xq_h = (x0_h, x2_h) if p == 0 else (x1_h, x3_h)
            pltpu.sync_copy(zeros_h.at[pl.ds(sid * rpt, rpt)],
                            acc.at[pl.ds(sid * rpt, rpt)])
            plsc.subcore_barrier()

            def load_sidx(j, b):
                base = sid * ept + j * _CH
                pltpu.sync_copy(srci_h.at[pl.ds(base, _CH)], sidxs[b])

            def load_didx(j, b):
                base = sid * ept + j * _CH
                pltpu.sync_copy(dsti_h.at[pl.ds(base, _CH)], didxs[b])

            def issue(b, q0_h, q1_h):
                @pl.when(cid == 0)
                def _a():
                    pltpu.async_copy(q0_h.at[sidxs[b]], xgs[b], sems[b])

                @pl.when(cid == 1)
                def _b():
                    pltpu.async_copy(q1_h.at[sidxs[b]], xgs[b], sems[b])

            def wait(b, q0_h, q1_h):
                @pl.when(cid == 0)
                def _a():
                    pltpu.make_async_copy(q0_h.at[sidxs[b]], xgs[b],
                                          sems[b]).wait()

                @pl.when(cid == 1)
                def _b():
                    pltpu.make_async_copy(q1_h.at[sidxs[b]], xgs[b],
                                          sems[b]).wait()

            def issue_ex(j, b):
                base = sid * ept + j * _CH
                pltpu.async_copy(ex_h.at[pl.ds(base, _CH)], exgs[b],
                                 esems[b])

            def wait_ex(j, b):
                base = sid * ept + j * _CH
                pltpu.make_async_copy(ex_h.at[pl.ds(base, _CH)], exgs[b],
                                      esems[b]).wait()

            q0_h, q1_h = xq_h
            load_sidx(0, 0)
            issue(0, q0_h, q1_h)
            issue_ex(0, 0)
            load_sidx(1, 1)
            issue(1, q0_h, q1_h)
            issue_ex(1, 1)

            def body(j2, carry):
                for b in range(2):
                    j = j2 * 2 + b
                    base = sid * ept + j * _CH
                    wait(b, q0_h, q1_h)
                    wait_ex(j, b)
                    exg = exgs[b]
                    xg = xgs[b]
                    scb = scbs[b]

                    @pl.when(j >= 2)
                    def _drain():
                        pltpu.make_async_copy(scb, acc.at[didxs[b]],
                                              ssems[b]).wait()

                    @plsc.parallel_loop(0, _CH, 1, unroll=8)
                    def inner(e):
                        exv = exg[e, :]
                        v0 = xg[e, pl.ds(0, 16)]
                        v1 = xg[e, pl.ds(16, 16)]
                        for hh in range(4):
                            w = exv[hh]
                            scb[e, pl.ds(hh * 32, 16)] = v0 * w
                            scb[e, pl.ds(hh * 32 + 16, 16)] = v1 * w

                    load_didx(j, b)
                    pltpu.async_copy(scb, acc.at[didxs[b]], ssems[b],
                                     add=True)

                    @pl.when(j + 2 < cpt)
                    def _next():
                        load_sidx(j + 2, b)
                        issue(b, q0_h, q1_h)
                        issue_ex(j + 2, b)
                return carry

            lax.fori_loop(0, cpt // 2, body, 0)
            for b in range(2):
                pltpu.make_async_copy(scbs[b], acc.at[didxs[b]],
                                      ssems[b]).wait()
            plsc.subcore_barrier()
            pltpu.sync_copy(acc.at[pl.ds(sid * rpt, rpt)],
                            out_h.at[cid, p, pl.ds(sid * rpt, rpt)])
            plsc.subcore_barrier()

    return k(xq0, xq1, xq2, xq3, ex, srci, dsti, zeros128)


def _gcn_dense_body(p_ref, dinv_ref, w_ref, b_ref, out_ref):
    s = (p_ref[0] + p_ref[1]) * dinv_ref[...]
    h = jnp.dot(s, w_ref[...], preferred_element_type=jnp.float32)
    h = jnp.maximum(h + b_ref[...], 0.0)
    out_ref[...] = h * dinv_ref[...]


def _gcn_dense(p, dinv, w, b, blk=1280):
    npad, fin = p.shape[1], p.shape[2]
    fout = w.shape[1]
    return pl.pallas_call(
        _gcn_dense_body,
        grid=(npad // blk,),
        in_specs=[
            pl.BlockSpec((2, blk, fin), lambda i: (0, i, 0)),
            pl.BlockSpec((blk, 1), lambda i: (i, 0)),
            pl.BlockSpec((fin, fout), lambda i: (0, 0)),
            pl.BlockSpec((1, fout), lambda i: (0, 0)),
        ],
        out_specs=pl.BlockSpec((blk, fout), lambda i: (i, 0)),
        out_shape=jax.ShapeDtypeStruct((npad, fout), jnp.float32),
    )(p, dinv, w, b)


def _gat_prep_body(p_ref, dinv_ref, w_ref, b_ref, avs_ref, avd_ref,
                   x2_ref, as_ref, ad_ref, c_ref):
    s = (p_ref[0] + p_ref[1]) * dinv_ref[...]
    x2 = jnp.dot(s, w_ref[...], preferred_element_type=jnp.float32)
    x2 = jnp.maximum(x2 + b_ref[...], 0.0)
    x2_ref[...] = x2
    a_s = jnp.dot(x2, avs_ref[...], preferred_element_type=jnp.float32)
    a_d = jnp.dot(x2, avd_ref[...], preferred_element_type=jnp.float32)
    pre = a_s + a_d
    c = jnp.maximum(pre, 0.2 * pre)
    z = jnp.zeros((a_s.shape[0], 12), jnp.float32)
    as_ref[...] = jnp.concatenate([a_s, z], axis=1)
    ad_ref[...] = jnp.concatenate([a_d, z], axis=1)
    c_ref[...] = jnp.concatenate([c, z], axis=1)


def _gat_prep(p, dinv, w, b, avs, avd, blk=1280):
    npad, fin = p.shape[1], p.shape[2]
    fout = w.shape[1]
    return pl.pallas_call(
        _gat_prep_body,
        grid=(npad // blk,),
        in_specs=[
            pl.BlockSpec((2, blk, fin), lambda i: (0, i, 0)),
            pl.BlockSpec((blk, 1), lambda i: (i, 0)),
            pl.BlockSpec((fin, fout), lambda i: (0, 0)),
            pl.BlockSpec((1, fout), lambda i: (0, 0)),
            pl.BlockSpec((fout, 4), lambda i: (0, 0)),
            pl.BlockSpec((fout, 4), lambda i: (0, 0)),
        ],
        out_specs=[
            pl.BlockSpec((blk, fout), lambda i: (i, 0)),
            pl.BlockSpec((blk, 16), lambda i: (i, 0)),
            pl.BlockSpec((blk, 16), lambda i: (i, 0)),
            pl.BlockSpec((blk, 16), lambda i: (i, 0)),
        ],
        out_shape=[
            jax.ShapeDtypeStruct((npad, fout), jnp.float32),
            jax.ShapeDtypeStruct((npad, 16), jnp.float32),
            jax.ShapeDtypeStruct((npad, 16), jnp.float32),
            jax.ShapeDtypeStruct((npad, 16), jnp.float32),
        ],
    )(p, dinv, w, b, avs, avd)


def _tail_body(a3_ref, dp_ref, w3s_ref, b3_ref, gw1_ref, gb1_ref, gw2_ref,
               gb2_ref, mw_ref, mb_ref, h_ref, g_ref, m_s, s_s, g_s):
    i = pl.program_id(0)
    nblk = pl.num_programs(0)

    @pl.when(i == 0)
    def _init():
        m_s[0, 0] = _NEG_INF
        s_s[0, 0] = 0.0
        g_s[...] = jnp.zeros_like(g_s)

    d = dp_ref[0] + dp_ref[1]
    r = 1.0 / (d + 1e-16)
    hid = h_ref.shape[1]
    acc = jnp.zeros((a3_ref.shape[1], hid), jnp.float32)
    for hh in range(4):
        sc = a3_ref[hh] * r[:, hh:hh + 1]
        acc = acc + jnp.dot(sc, w3s_ref[hh * 128:(hh + 1) * 128, :],
                            preferred_element_type=jnp.float32)
    h = acc * 0.25 + b3_ref[...]
    h_ref[...] = h

    z1 = jnp.maximum(
        jnp.dot(h, gw1_ref[...], preferred_element_type=jnp.float32)
        + gb1_ref[...], 0.0)
    z = jnp.dot(z1, gw2_ref[...], preferred_element_type=jnp.float32)
    z = z + gb2_ref[0, 0]

    blk_max = jnp.max(z)
    m_old = m_s[0, 0]
    m_new = jnp.maximum(m_old, blk_max)
    corr = jnp.exp(m_old - m_new)
    p = jnp.exp(z - m_new)
    s_s[0, 0] = s_s[0, 0] * corr + jnp.sum(p)
    g_s[...] = g_s[...] * corr + jnp.sum(p * h, axis=0, keepdims=True)
    m_s[0, 0] = m_new

    @pl.when(i == nblk - 1)
    def _fin():
        g = g_s[...] / s_s[0, 0]
        g = jnp.dot(g, mw_ref[...], preferred_element_type=jnp.float32)
        g_ref[...] = jnp.maximum(g + mb_ref[...], 0.0)


def _tail(a3, dp, w3s, b3, gw1, gb1, gw2, gb2, mw, mb, n, blk=400):
    hid = w3s.shape[1]
    return pl.pallas_call(
        _tail_body,
        grid=(n // blk,),
        in_specs=[
            pl.BlockSpec((4, blk, 128), lambda i: (0, i, 0)),
            pl.BlockSpec((2, blk, 16), lambda i: (0, i, 0)),
            pl.BlockSpec((512, hid), lambda i: (0, 0)),
            pl.BlockSpec((1, hid), lambda i: (0, 0)),
            pl.BlockSpec((hid, hid), lambda i: (0, 0)),
            pl.BlockSpec((1, hid), lambda i: (0, 0)),
            pl.BlockSpec((hid, 1), lambda i: (0, 0)),
            pl.BlockSpec((1, 1), lambda i: (0, 0)),
            pl.BlockSpec((hid, hid), lambda i: (0, 0)),
            pl.BlockSpec((1, hid), lambda i: (0, 0)),
        ],
        out_specs=[
            pl.BlockSpec((blk, hid), lambda i: (i, 0)),
            pl.BlockSpec((1, hid), lambda i: (0, 0)),
        ],
        out_shape=[
            jax.ShapeDtypeStruct((n, hid), jnp.float32),
            jax.ShapeDtypeStruct((1, hid), jnp.float32),
        ],
        scratch_shapes=[
            pltpu.SMEM((1, 1), jnp.float32),
            pltpu.SMEM((1, 1), jnp.float32),
            pltpu.VMEM((1, hid), jnp.float32),
        ],
    )(a3, dp, w3s, b3, gw1, gb1, gw2, gb2, mw, mb)


def kernel(x, edge_index, W1, b1, W2, b2, W3, att_src, att_dst, b3,
           gate_W1, gate_b1, gate_W2, gate_b2, mlp_W, mlp_b):
    n = x.shape[0]
    heads, hidden = att_src.shape
    in2 = W3.shape[0]
    npad = _NPAD

    loop = jnp.arange(n, dtype=jnp.int32)
    src = jnp.concatenate([edge_index[0], loop])
    dst = jnp.concatenate([edge_index[1], loop])
    e_real = src.shape[0]
    etot = ((e_real + 4095) // 4096) * 4096
    padn = etot - e_real
    srcp = jnp.concatenate([src, jnp.zeros((padn,), jnp.int32)])
    dstp = jnp.concatenate([dst, jnp.full((padn,), n, jnp.int32)])

    zeros16 = jnp.zeros((npad, 16), jnp.float32)
    zeros64 = jnp.zeros((npad, 64), jnp.float32)
    ones16 = jnp.ones((_CH, 16), jnp.float32)

    degp = _deg_sc(dstp, ones16, zeros16, etot)
    deg = degp[0, :, 0] + degp[1, :, 0]
    dinv = lax.rsqrt(jnp.maximum(deg, 1e-12))
    dinv2 = dinv[:, None]

    xs16 = jnp.zeros((npad, 16), jnp.float32)
    xs16 = xs16.at[:n, :3].set(x * dinv2[:n])
    agg1 = _segsum_sc(xs16, srcp, dstp, zeros16, 16, etot)
    w1p = jnp.zeros((16, 64), jnp.float32).at[:3].set(W1)
    x1s = _gcn_dense(agg1, dinv2, w1p, b1.reshape(1, 64))

    agg2 = _segsum_sc(x1s, srcp, dstp, zeros64, 64, etot)
    w3r = W3.reshape(in2, heads, hidden)
    avs = jnp.einsum("khj,hj->kh", w3r, att_src)
    avd = jnp.einsum("khj,hj->kh", w3r, att_dst)
    x2, asT, adT, cT = _gat_prep(agg2, dinv2, W2, b2.reshape(1, in2),
                                 avs, avd)

    exT, dpart = _gat_edge_sc(asT, adT, cT, srcp, dstp, zeros16, etot)
    zeros128 = jnp.zeros((npad, 128), jnp.float32)
    xq = [x2[:, 32 * q:32 * (q + 1)] for q in range(4)]
    aq = _gat_agg_sc(xq[0], xq[1], xq[2], xq[3], exT, srcp, dstp,
                     zeros128, etot)
    a3 = aq.reshape(4, npad, 4, 32).transpose(2, 1, 0, 3).reshape(
        4, npad, 128)

    w3s = w3r.transpose(1, 0, 2).reshape(heads * in2, hidden)
    h, g = _tail(a3, dpart, w3s, b3.reshape(1, hidden),
                 gate_W1, gate_b1.reshape(1, hidden),
                 gate_W2, gate_b2.reshape(1, 1),
                 mlp_W, mlp_b.reshape(1, hidden), n)
    return (g, h)
